# Initial kernel scaffold; baseline (speedup 1.0000x reference)
import jax
import jax.numpy as jnp
from jax.experimental import pallas as pl


def kernel(x, edge_index, edge_type, weight, alpha, bias, weight_self_loop, ln_gamma, ln_beta):
    raise NotImplementedError("write your pallas kernel here")


# speedup vs baseline: 17.9342x; 17.9342x over previous
import functools

import jax
import jax.numpy as jnp
from jax import lax
from jax.experimental import pallas as pl
from jax.experimental.pallas import tpu as pltpu
from jax.experimental.pallas import tpu_sc as plsc


def _dense_body(x_ref, w_ref, a_ref, ws_ref, h_ref, s_ref):
    xb = x_ref[...]
    nb = w_ref.shape[0]
    r = h_ref.shape[0]
    hb = [jnp.dot(xb, w_ref[b], preferred_element_type=jnp.float32)
          for b in range(nb)]
    for i in range(r):
        acc = a_ref[i, 0] * hb[0]
        for b in range(1, nb):
            acc = acc + a_ref[i, b] * hb[b]
        h_ref[i] = acc
    s_ref[...] = jnp.dot(xb, ws_ref[...], preferred_element_type=jnp.float32)


def _dense(x, weight, alpha, w_self, bn):
    n, d = x.shape
    nb, _, do = weight.shape
    r = alpha.shape[0]
    grid = (n // bn,)
    return pl.pallas_call(
        _dense_body,
        grid=grid,
        in_specs=[
            pl.BlockSpec((bn, d), lambda i: (i, 0)),
            pl.BlockSpec((nb, d, do), lambda i: (0, 0, 0)),
            pl.BlockSpec(memory_space=pltpu.SMEM),
            pl.BlockSpec((d, do), lambda i: (0, 0)),
        ],
        out_specs=[
            pl.BlockSpec((r, bn, do), lambda i: (0, i, 0)),
            pl.BlockSpec((bn, do), lambda i: (i, 0)),
        ],
        out_shape=[
            jax.ShapeDtypeStruct((r, n, do), jnp.float32),
            jax.ShapeDtypeStruct((n, do), jnp.float32),
        ],
    )(x, weight, alpha, w_self)


def _make_scatter(n, e, d, r):
    info = plsc.get_sparse_core_info()
    ncores, nsub, lanes = info.num_cores, info.num_subcores, info.num_lanes
    nw = ncores * nsub
    epw = e // nw
    k = 80
    nchunk = epw // k
    rpt = n // nsub
    zr = 125
    assert epw % k == 0 and rpt % zr == 0 and k % lanes == 0

    mesh = plsc.VectorSubcoreMesh(core_axis_name="c", subcore_axis_name="s")

    @functools.partial(
        pl.kernel,
        out_type=(
            jax.ShapeDtypeStruct((ncores, n, d), jnp.float32),
            jax.ShapeDtypeStruct((ncores * n,), jnp.float32),
        ),
        mesh=mesh,
        scratch_types=[
            pltpu.VMEM((k,), jnp.int32),
            pltpu.VMEM((k,), jnp.int32),
            pltpu.VMEM((k,), jnp.int32),
            pltpu.VMEM((k,), jnp.int32),
            pltpu.VMEM((k, d), jnp.float32),
            pltpu.VMEM((k,), jnp.float32),
            pltpu.VMEM((zr, d), jnp.float32),
            pltpu.VMEM((640,), jnp.float32),
            pltpu.VMEM((104, d), jnp.float32),
            pltpu.VMEM((624,), jnp.float32),
            pltpu.VMEM_SHARED((n, d), jnp.float32),
            pltpu.VMEM_SHARED((n,), jnp.float32),
            pltpu.SemaphoreType.DMA,
        ],
    )
    def sc_scatter(row_hbm, col_hbm, typ_hbm, h_hbm, out_hbm, deg_hbm,
                   row_buf, col_buf, typ_buf, gidx_buf, rows_buf, ones_buf,
                   ztile, zvec, bounce, dbounce, accum, dega, sem):
        c = lax.axis_index("c")
        s = lax.axis_index("s")
        wid = s * ncores + c

        zero16 = jnp.zeros((lanes,), jnp.float32)
        one16 = jnp.ones((lanes,), jnp.float32)
        for i in range(k // lanes):
            ones_buf[pl.ds(i * lanes, lanes)] = one16

        def zrow(i, _):
            for j in range(d // lanes):
                ztile[i, pl.ds(j * lanes, lanes)] = zero16
            return 0
        lax.fori_loop(0, zr, zrow, 0)

        def zv(i, _):
            zvec[pl.ds(i * lanes, lanes)] = zero16
            return 0
        lax.fori_loop(0, 640 // lanes, zv, 0)

        for m in range(rpt // zr):
            pltpu.sync_copy(ztile, accum.at[pl.ds(s * rpt + m * zr, zr)])
        pltpu.sync_copy(zvec.at[pl.ds(0, 624)], dega.at[pl.ds(s * 624, 624)])

        @pl.when(s == 0)
        def _():
            pltpu.sync_copy(zvec.at[pl.ds(0, 16)], dega.at[pl.ds(9984, 16)])
        plsc.subcore_barrier()

        def chunk(j, _):
            base = wid * epw + j * k
            pltpu.sync_copy(row_hbm.at[pl.ds(base, k)], row_buf)
            pltpu.sync_copy(col_hbm.at[pl.ds(base, k)], col_buf)
            pltpu.sync_copy(typ_hbm.at[pl.ds(base, k)], typ_buf)
            for i in range(k // lanes):
                sl = pl.ds(i * lanes, lanes)
                gidx_buf[sl] = typ_buf[sl] * n + col_buf[sl]
            pltpu.async_copy(h_hbm.at[gidx_buf], rows_buf, sem).wait()
            pltpu.sync_copy(rows_buf, accum.at[row_buf], add=True)
            pltpu.sync_copy(ones_buf, dega.at[row_buf], add=True)
            return 0
        lax.fori_loop(0, nchunk, chunk, 0)
        plsc.subcore_barrier()

        for m in range(6):
            off = s * 624 + m * 104
            pltpu.sync_copy(accum.at[pl.ds(off, 104)], bounce)
            pltpu.sync_copy(bounce, out_hbm.at[c, pl.ds(off, 104)])
        pltpu.sync_copy(dega.at[pl.ds(s * 624, 624)], dbounce)
        pltpu.sync_copy(dbounce, deg_hbm.at[pl.ds(c * n + s * 624, 624)])

        @pl.when(s == 0)
        def _():
            pltpu.sync_copy(accum.at[pl.ds(9984, 16)],
                            bounce.at[pl.ds(0, 16)])
            pltpu.sync_copy(bounce.at[pl.ds(0, 16)],
                            out_hbm.at[c, pl.ds(9984, 16)])
            pltpu.sync_copy(dega.at[pl.ds(9984, 16)],
                            dbounce.at[pl.ds(0, 16)])
            pltpu.sync_copy(dbounce.at[pl.ds(0, 16)],
                            deg_hbm.at[pl.ds(c * n + 9984, 16)])

    return sc_scatter


def _finalize_body(p_ref, dg_ref, sf_ref, b_ref, g_ref, be_ref, o_ref):
    ssum = p_ref[0] + p_ref[1]
    deg = dg_ref[0] + dg_ref[1]
    recip = jnp.where(deg > 0, 1.0 / deg, jnp.zeros_like(deg))
    h = ssum * recip
    mean = jnp.mean(h, axis=-1, keepdims=True)
    var = jnp.mean((h - mean) * (h - mean), axis=-1, keepdims=True)
    hn = (h - mean) * lax.rsqrt(var + 1e-5)
    o_ref[...] = hn * g_ref[...] + be_ref[...] + b_ref[...] + sf_ref[...]


def _finalize(part, degp, selfx, bias, gamma, beta, bn):
    nc, n, d = part.shape
    grid = (n // bn,)
    return pl.pallas_call(
        _finalize_body,
        grid=grid,
        in_specs=[
            pl.BlockSpec((nc, bn, d), lambda i: (0, i, 0)),
            pl.BlockSpec((nc, bn, 1), lambda i: (0, i, 0)),
            pl.BlockSpec((bn, d), lambda i: (i, 0)),
            pl.BlockSpec((1, d), lambda i: (0, 0)),
            pl.BlockSpec((1, d), lambda i: (0, 0)),
            pl.BlockSpec((1, d), lambda i: (0, 0)),
        ],
        out_specs=pl.BlockSpec((bn, d), lambda i: (i, 0)),
        out_shape=jax.ShapeDtypeStruct((n, d), jnp.float32),
    )(part, degp, selfx, bias, gamma, beta)


def kernel(x, edge_index, edge_type, weight, alpha, bias, weight_self_loop,
           ln_gamma, ln_beta):
    n, d = x.shape
    e = edge_type.shape[0]
    r = alpha.shape[0]
    do = weight.shape[2]
    bn = 400

    row = edge_index[0]
    col = edge_index[1]

    h_all, selfx = _dense(x, weight, alpha, weight_self_loop, bn)
    h_flat = h_all.reshape(r * n, do)

    part, degp = _make_scatter(n, e, do, r)(row, col, edge_type, h_flat)
    degp = degp.reshape(2, n)

    out = _finalize(part, degp[..., None], selfx,
                    bias.reshape(1, do), ln_gamma.reshape(1, do),
                    ln_beta.reshape(1, do), bn)
    return out


# speedup vs baseline: 35.6183x; 1.9861x over previous
import functools

import jax
import jax.numpy as jnp
from jax import lax
from jax.experimental import pallas as pl
from jax.experimental.pallas import tpu as pltpu
from jax.experimental.pallas import tpu_sc as plsc


def _dense_body(x_ref, w_ref, a_ref, ws_ref, h_ref, s_ref):
    xb = x_ref[...]
    nb = w_ref.shape[0]
    r = h_ref.shape[0]
    hb = [jnp.dot(xb, w_ref[b], preferred_element_type=jnp.float32)
          for b in range(nb)]
    for i in range(r):
        acc = a_ref[i, 0] * hb[0]
        for b in range(1, nb):
            acc = acc + a_ref[i, b] * hb[b]
        h_ref[i] = acc
    s_ref[...] = jnp.dot(xb, ws_ref[...], preferred_element_type=jnp.float32)


def _dense(x, weight, alpha, w_self, bn):
    n, d = x.shape
    nb, _, do = weight.shape
    r = alpha.shape[0]
    grid = (n // bn,)
    return pl.pallas_call(
        _dense_body,
        grid=grid,
        in_specs=[
            pl.BlockSpec((bn, d), lambda i: (i, 0)),
            pl.BlockSpec((nb, d, do), lambda i: (0, 0, 0)),
            pl.BlockSpec(memory_space=pltpu.SMEM),
            pl.BlockSpec((d, do), lambda i: (0, 0)),
        ],
        out_specs=[
            pl.BlockSpec((r, bn, do), lambda i: (0, i, 0)),
            pl.BlockSpec((bn, do), lambda i: (i, 0)),
        ],
        out_shape=[
            jax.ShapeDtypeStruct((r, n, do), jnp.float32),
            jax.ShapeDtypeStruct((n, do), jnp.float32),
        ],
    )(x, weight, alpha, w_self)


def _make_scatter(n, e, d, r):
    info = plsc.get_sparse_core_info()
    ncores, nsub, lanes = info.num_cores, info.num_subcores, info.num_lanes
    nw = ncores * nsub
    epw = e // nw
    k = 80
    nchunk = epw // k
    g = 25
    nsup = nchunk // g
    rpt = n // nsub
    assert epw % k == 0 and nchunk % g == 0 and k % lanes == 0

    mesh = plsc.VectorSubcoreMesh(core_axis_name="c", subcore_axis_name="s")

    @functools.partial(
        pl.kernel,
        out_type=(
            jax.ShapeDtypeStruct((ncores, n, d), jnp.float32),
            jax.ShapeDtypeStruct((ncores * n,), jnp.float32),
        ),
        mesh=mesh,
        scratch_types=[
            pltpu.VMEM((1, 1, g, k), jnp.int32),
            pltpu.VMEM((1, 1, g, k), jnp.int32),
            pltpu.VMEM((1, 1, g, k), jnp.int32),
            pltpu.VMEM((k, d), jnp.float32),
            pltpu.VMEM((k, d), jnp.float32),
            pltpu.VMEM((k,), jnp.float32),
            pltpu.VMEM((640,), jnp.float32),
            pltpu.VMEM((624,), jnp.float32),
            pltpu.VMEM_SHARED((n, d), jnp.float32),
            pltpu.VMEM_SHARED((n,), jnp.float32),
            pltpu.SemaphoreType.DMA,
            pltpu.SemaphoreType.DMA,
        ],
    )
    def sc_scatter(row_hbm, col_hbm, typ_hbm, h_hbm, out_hbm, deg_hbm,
                   row_all, gidx_all, col_all, rows0, rows1, ones_buf,
                   zvec, dbounce, accum, dega, sem0, sem1):
        c = lax.axis_index("c")
        s = lax.axis_index("s")
        wid = s * ncores + c

        zero16 = jnp.zeros((lanes,), jnp.float32)
        one16 = jnp.ones((lanes,), jnp.float32)
        for i in range(k // lanes):
            ones_buf[pl.ds(i * lanes, lanes)] = one16

        def zrow(i, _):
            for j in range(d // lanes):
                rows0[i, pl.ds(j * lanes, lanes)] = zero16
            return 0
        lax.fori_loop(0, k, zrow, 0)

        def zv(i, _):
            zvec[pl.ds(i * lanes, lanes)] = zero16
            return 0
        lax.fori_loop(0, 640 // lanes, zv, 0)

        nz = rpt // k
        rem = rpt - nz * k
        for m in range(nz):
            pltpu.sync_copy(rows0, accum.at[pl.ds(s * rpt + m * k, k)])
        if rem:
            pltpu.sync_copy(rows0.at[pl.ds(0, rem)],
                            accum.at[pl.ds(s * rpt + nz * k, rem)])
        pltpu.sync_copy(zvec.at[pl.ds(0, 624)], dega.at[pl.ds(s * 624, 624)])

        @pl.when(s == 0)
        def _():
            pltpu.sync_copy(zvec.at[pl.ds(0, 16)], dega.at[pl.ds(9984, 16)])
        plsc.subcore_barrier()

        def issue(j, buf, sem):
            pltpu.async_copy(h_hbm.at[gidx_all.at[0, 0, j]], buf, sem)

        def consume(j, buf, sem):
            pltpu.make_async_copy(h_hbm.at[gidx_all.at[0, 0, j]], buf,
                                  sem).wait()
            pltpu.sync_copy(buf, accum.at[row_all.at[0, 0, j]], add=True)
            pltpu.sync_copy(ones_buf, dega.at[row_all.at[0, 0, j]], add=True)

        def sup_body(sup, _):
            pltpu.sync_copy(row_hbm.at[pl.ds(wid, 1), pl.ds(sup, 1)], row_all)
            pltpu.sync_copy(typ_hbm.at[pl.ds(wid, 1), pl.ds(sup, 1)],
                            gidx_all)
            pltpu.sync_copy(col_hbm.at[pl.ds(wid, 1), pl.ds(sup, 1)], col_all)

            def gj(j, _):
                for i in range(k // lanes):
                    sl = pl.ds(i * lanes, lanes)
                    gidx_all[0, 0, j, sl] = (gidx_all[0, 0, j, sl] * n
                                             + col_all[0, 0, j, sl])
                return 0
            lax.fori_loop(0, g, gj, 0)

            issue(0, rows0, sem0)
            issue(1, rows1, sem1)

            def pair(jj, _):
                j0 = jj * 2
                j1 = j0 + 1
                consume(j0, rows0, sem0)

                @pl.when(j0 + 2 < g)
                def _():
                    issue(j0 + 2, rows0, sem0)

                @pl.when(j1 < g)
                def _():
                    consume(j1, rows1, sem1)

                    @pl.when(j1 + 2 < g)
                    def _():
                        issue(j1 + 2, rows1, sem1)
                return 0
            lax.fori_loop(0, (g + 1) // 2, pair, 0)
            return 0
        lax.fori_loop(0, nsup, sup_body, 0)
        plsc.subcore_barrier()

        for m in range(7):
            off = s * 624 + m * k
            buf = rows0 if m % 2 == 0 else rows1
            pltpu.sync_copy(accum.at[pl.ds(off, k)], buf)
            pltpu.sync_copy(buf, out_hbm.at[c, pl.ds(off, k)])
        off = s * 624 + 7 * k
        pltpu.sync_copy(accum.at[pl.ds(off, 64)], rows1.at[pl.ds(0, 64)])
        pltpu.sync_copy(rows1.at[pl.ds(0, 64)], out_hbm.at[c, pl.ds(off, 64)])
        pltpu.sync_copy(dega.at[pl.ds(s * 624, 624)], dbounce)
        pltpu.sync_copy(dbounce, deg_hbm.at[pl.ds(c * n + s * 624, 624)])

        @pl.when(s == 0)
        def _():
            pltpu.sync_copy(accum.at[pl.ds(9984, 16)], rows0.at[pl.ds(0, 16)])
            pltpu.sync_copy(rows0.at[pl.ds(0, 16)],
                            out_hbm.at[c, pl.ds(9984, 16)])
            pltpu.sync_copy(dega.at[pl.ds(9984, 16)], dbounce.at[pl.ds(0, 16)])
            pltpu.sync_copy(dbounce.at[pl.ds(0, 16)],
                            deg_hbm.at[pl.ds(c * n + 9984, 16)])

    return sc_scatter


def _finalize_body(p_ref, dg_ref, sf_ref, b_ref, g_ref, be_ref, o_ref):
    ssum = p_ref[0] + p_ref[1]
    deg = dg_ref[0] + dg_ref[1]
    recip = jnp.where(deg > 0, 1.0 / deg, jnp.zeros_like(deg))
    h = ssum * recip
    mean = jnp.mean(h, axis=-1, keepdims=True)
    var = jnp.mean((h - mean) * (h - mean), axis=-1, keepdims=True)
    hn = (h - mean) * lax.rsqrt(var + 1e-5)
    o_ref[...] = hn * g_ref[...] + be_ref[...] + b_ref[...] + sf_ref[...]


def _finalize(part, degp, selfx, bias, gamma, beta, bn):
    nc, n, d = part.shape
    grid = (n // bn,)
    return pl.pallas_call(
        _finalize_body,
        grid=grid,
        in_specs=[
            pl.BlockSpec((nc, bn, d), lambda i: (0, i, 0)),
            pl.BlockSpec((nc, bn, 1), lambda i: (0, i, 0)),
            pl.BlockSpec((bn, d), lambda i: (i, 0)),
            pl.BlockSpec((1, d), lambda i: (0, 0)),
            pl.BlockSpec((1, d), lambda i: (0, 0)),
            pl.BlockSpec((1, d), lambda i: (0, 0)),
        ],
        out_specs=pl.BlockSpec((bn, d), lambda i: (i, 0)),
        out_shape=jax.ShapeDtypeStruct((n, d), jnp.float32),
    )(part, degp, selfx, bias, gamma, beta)


def kernel(x, edge_index, edge_type, weight, alpha, bias, weight_self_loop,
           ln_gamma, ln_beta):
    n, d = x.shape
    e = edge_type.shape[0]
    r = alpha.shape[0]
    do = weight.shape[2]
    bn = 400

    nw, k, g = 32, 80, 25
    nsup = e // (nw * g * k)
    row = edge_index[0].reshape(nw, nsup, g, k)
    col = edge_index[1].reshape(nw, nsup, g, k)
    typ = edge_type.reshape(nw, nsup, g, k)

    h_all, selfx = _dense(x, weight, alpha, weight_self_loop, bn)
    h_flat = h_all.reshape(r * n, do)

    part, degp = _make_scatter(n, e, do, r)(row, col, typ, h_flat)
    degp = degp.reshape(2, n)

    out = _finalize(part, degp[..., None], selfx,
                    bias.reshape(1, do), ln_gamma.reshape(1, do),
                    ln_beta.reshape(1, do), bn)
    return out


# speedup vs baseline: 35.8369x; 1.0061x over previous
import functools

import jax
import jax.numpy as jnp
from jax import lax
from jax.experimental import pallas as pl
from jax.experimental.pallas import tpu as pltpu
from jax.experimental.pallas import tpu_sc as plsc


def _dense_body(x_ref, w_ref, a_ref, ws_ref, h_ref, s_ref):
    xb = x_ref[...]
    nb = w_ref.shape[0]
    r = h_ref.shape[0]
    hb = [jnp.dot(xb, w_ref[b], preferred_element_type=jnp.float32)
          for b in range(nb)]
    for i in range(r):
        acc = a_ref[i, 0] * hb[0]
        for b in range(1, nb):
            acc = acc + a_ref[i, b] * hb[b]
        h_ref[i] = acc
    s_ref[...] = jnp.dot(xb, ws_ref[...], preferred_element_type=jnp.float32)


def _dense(x, weight, alpha, w_self, bn):
    n, d = x.shape
    nb, _, do = weight.shape
    r = alpha.shape[0]
    grid = (n // bn,)
    return pl.pallas_call(
        _dense_body,
        grid=grid,
        in_specs=[
            pl.BlockSpec((bn, d), lambda i: (i, 0)),
            pl.BlockSpec((nb, d, do), lambda i: (0, 0, 0)),
            pl.BlockSpec(memory_space=pltpu.SMEM),
            pl.BlockSpec((d, do), lambda i: (0, 0)),
        ],
        out_specs=[
            pl.BlockSpec((r, bn, do), lambda i: (0, i, 0)),
            pl.BlockSpec((bn, do), lambda i: (i, 0)),
        ],
        out_shape=[
            jax.ShapeDtypeStruct((r, n, do), jnp.float32),
            jax.ShapeDtypeStruct((n, do), jnp.float32),
        ],
    )(x, weight, alpha, w_self)


def _make_scatter(n, e, d):
    info = plsc.get_sparse_core_info()
    ncores, nsub, lanes = info.num_cores, info.num_subcores, info.num_lanes
    nw = ncores * nsub
    epw = e // nw
    k = 128
    g = 13
    nsup = 6
    tail = epw - nsup * g * k
    rpt = n // nsub
    assert tail == 16 and k % lanes == 0

    mesh = plsc.VectorSubcoreMesh(core_axis_name="c", subcore_axis_name="s")

    @functools.partial(
        pl.kernel,
        out_type=(
            jax.ShapeDtypeStruct((ncores, n, d), jnp.float32),
            jax.ShapeDtypeStruct((ncores * n,), jnp.float32),
        ),
        mesh=mesh,
        scratch_types=[
            pltpu.VMEM((1, 1, g, k), jnp.int32),
            pltpu.VMEM((1, 1, g, k), jnp.int32),
            pltpu.VMEM((1, 1, g, k), jnp.int32),
            pltpu.VMEM((1, 1, tail), jnp.int32),
            pltpu.VMEM((1, 1, tail), jnp.int32),
            pltpu.VMEM((1, 1, tail), jnp.int32),
            pltpu.VMEM((k, d), jnp.float32),
            pltpu.VMEM((k, d), jnp.float32),
            pltpu.VMEM((k,), jnp.float32),
            pltpu.VMEM((640,), jnp.float32),
            pltpu.VMEM((624,), jnp.float32),
            pltpu.VMEM_SHARED((n, d), jnp.float32),
            pltpu.VMEM_SHARED((n,), jnp.float32),
            pltpu.SemaphoreType.DMA,
            pltpu.SemaphoreType.DMA,
            pltpu.SemaphoreType.DMA,
            pltpu.SemaphoreType.DMA,
        ],
    )
    def sc_scatter(rowm_hbm, colm_hbm, typm_hbm, rowt_hbm, colt_hbm,
                   typt_hbm, h_hbm, out_hbm, deg_hbm,
                   row_all, gidx_all, col_all, row_t, gidx_t, col_t,
                   rows0, rows1, ones_buf, zvec, dbounce, accum, dega,
                   semg0, semg1, sems0, sems1):
        c = lax.axis_index("c")
        s = lax.axis_index("s")
        wid = s * ncores + c

        zero16 = jnp.zeros((lanes,), jnp.float32)
        izero16 = jnp.zeros((lanes,), jnp.int32)
        one16 = jnp.ones((lanes,), jnp.float32)
        for i in range(k // lanes):
            ones_buf[pl.ds(i * lanes, lanes)] = one16
        for i in range(k // lanes):
            row_all[0, 0, 0, pl.ds(i * lanes, lanes)] = izero16

        def zrow(i, _):
            for j in range(d // lanes):
                rows0[i, pl.ds(j * lanes, lanes)] = zero16
                rows1[i, pl.ds(j * lanes, lanes)] = zero16
            return 0
        lax.fori_loop(0, k, zrow, 0)

        def zv(i, _):
            zvec[pl.ds(i * lanes, lanes)] = zero16
            return 0
        lax.fori_loop(0, 640 // lanes, zv, 0)

        nz = rpt // k
        rem = rpt - nz * k
        for m in range(nz):
            pltpu.sync_copy(rows0, accum.at[pl.ds(s * rpt + m * k, k)])
        if rem:
            pltpu.sync_copy(rows0.at[pl.ds(0, rem)],
                            accum.at[pl.ds(s * rpt + nz * k, rem)])
        pltpu.sync_copy(zvec.at[pl.ds(0, 624)], dega.at[pl.ds(s * 624, 624)])

        @pl.when(s == 0)
        def _():
            pltpu.sync_copy(zvec.at[pl.ds(0, 16)], dega.at[pl.ds(9984, 16)])
        plsc.subcore_barrier()

        pltpu.async_copy(rows0, accum.at[row_all.at[0, 0, 0]], sems0,
                         add=True)
        pltpu.async_copy(ones_buf, dega.at[row_all.at[0, 0, 0]], sems0,
                         add=True)
        pltpu.async_copy(rows1, accum.at[row_all.at[0, 0, 0]], sems1,
                         add=True)
        pltpu.async_copy(ones_buf, dega.at[row_all.at[0, 0, 0]], sems1,
                         add=True)

        def issue(j, buf, semg, sems):
            pltpu.make_async_copy(buf, accum.at[row_all.at[0, 0, j]],
                                  sems).wait()
            pltpu.make_async_copy(ones_buf, dega.at[row_all.at[0, 0, j]],
                                  sems).wait()
            pltpu.async_copy(h_hbm.at[gidx_all.at[0, 0, j]], buf, semg)

        def consume(j, buf, semg, sems):
            pltpu.make_async_copy(h_hbm.at[gidx_all.at[0, 0, j]], buf,
                                  semg).wait()
            pltpu.async_copy(buf, accum.at[row_all.at[0, 0, j]], sems,
                             add=True)
            pltpu.async_copy(ones_buf, dega.at[row_all.at[0, 0, j]], sems,
                             add=True)

        def sup_body(sup, _):
            pltpu.sync_copy(rowm_hbm.at[pl.ds(wid, 1), pl.ds(sup, 1)],
                            row_all)
            pltpu.sync_copy(typm_hbm.at[pl.ds(wid, 1), pl.ds(sup, 1)],
                            gidx_all)
            pltpu.sync_copy(colm_hbm.at[pl.ds(wid, 1), pl.ds(sup, 1)],
                            col_all)

            def gj(j, _):
                for i in range(k // lanes):
                    sl = pl.ds(i * lanes, lanes)
                    gidx_all[0, 0, j, sl] = (gidx_all[0, 0, j, sl] * n
                                             + col_all[0, 0, j, sl])
                return 0
            lax.fori_loop(0, g, gj, 0)

            issue(0, rows0, semg0, sems0)
            issue(1, rows1, semg1, sems1)

            def pair(jj, _):
                j0 = jj * 2
                j1 = j0 + 1
                consume(j0, rows0, semg0, sems0)

                @pl.when(j0 + 2 < g)
                def _():
                    issue(j0 + 2, rows0, semg0, sems0)

                @pl.when(j1 < g)
                def _():
                    consume(j1, rows1, semg1, sems1)

                    @pl.when(j1 + 2 < g)
                    def _():
                        issue(j1 + 2, rows1, semg1, sems1)
                return 0
            lax.fori_loop(0, (g + 1) // 2, pair, 0)
            return 0
        lax.fori_loop(0, nsup, sup_body, 0)

        pltpu.make_async_copy(rows0, accum.at[row_all.at[0, 0, 0]],
                              sems0).wait()
        pltpu.make_async_copy(ones_buf, dega.at[row_all.at[0, 0, 0]],
                              sems0).wait()
        pltpu.make_async_copy(rows1, accum.at[row_all.at[0, 0, 0]],
                              sems1).wait()
        pltpu.make_async_copy(ones_buf, dega.at[row_all.at[0, 0, 0]],
                              sems1).wait()

        pltpu.sync_copy(rowt_hbm.at[pl.ds(wid, 1)], row_t)
        pltpu.sync_copy(typt_hbm.at[pl.ds(wid, 1)], gidx_t)
        pltpu.sync_copy(colt_hbm.at[pl.ds(wid, 1)], col_t)
        gidx_t[0, 0, :] = gidx_t[0, 0, :] * n + col_t[0, 0, :]
        pltpu.async_copy(h_hbm.at[gidx_t.at[0, 0]],
                         rows0.at[pl.ds(0, tail)], semg0).wait()
        pltpu.sync_copy(rows0.at[pl.ds(0, tail)],
                        accum.at[row_t.at[0, 0]], add=True)
        pltpu.sync_copy(ones_buf.at[pl.ds(0, tail)],
                        dega.at[row_t.at[0, 0]], add=True)
        plsc.subcore_barrier()

        for m in range(4):
            off = s * 624 + m * k
            buf = rows0 if m % 2 == 0 else rows1
            pltpu.sync_copy(accum.at[pl.ds(off, k)], buf)
            pltpu.sync_copy(buf, out_hbm.at[c, pl.ds(off, k)])
        off = s * 624 + 4 * k
        pltpu.sync_copy(accum.at[pl.ds(off, 112)], rows1.at[pl.ds(0, 112)])
        pltpu.sync_copy(rows1.at[pl.ds(0, 112)],
                        out_hbm.at[c, pl.ds(off, 112)])
        pltpu.sync_copy(dega.at[pl.ds(s * 624, 624)], dbounce)
        pltpu.sync_copy(dbounce, deg_hbm.at[pl.ds(c * n + s * 624, 624)])

        @pl.when(s == 0)
        def _():
            pltpu.sync_copy(accum.at[pl.ds(9984, 16)], rows0.at[pl.ds(0, 16)])
            pltpu.sync_copy(rows0.at[pl.ds(0, 16)],
                            out_hbm.at[c, pl.ds(9984, 16)])
            pltpu.sync_copy(dega.at[pl.ds(9984, 16)], dbounce.at[pl.ds(0, 16)])
            pltpu.sync_copy(dbounce.at[pl.ds(0, 16)],
                            deg_hbm.at[pl.ds(c * n + 9984, 16)])

    return sc_scatter


def _finalize_body(p_ref, dg_ref, sf_ref, b_ref, g_ref, be_ref, o_ref):
    ssum = p_ref[0] + p_ref[1]
    deg = dg_ref[0] + dg_ref[1]
    recip = jnp.where(deg > 0, 1.0 / deg, jnp.zeros_like(deg))
    h = ssum * recip
    mean = jnp.mean(h, axis=-1, keepdims=True)
    var = jnp.mean((h - mean) * (h - mean), axis=-1, keepdims=True)
    hn = (h - mean) * lax.rsqrt(var + 1e-5)
    o_ref[...] = hn * g_ref[...] + be_ref[...] + b_ref[...] + sf_ref[...]


def _finalize(part, degp, selfx, bias, gamma, beta, bn):
    nc, n, d = part.shape
    grid = (n // bn,)
    return pl.pallas_call(
        _finalize_body,
        grid=grid,
        in_specs=[
            pl.BlockSpec((nc, bn, d), lambda i: (0, i, 0)),
            pl.BlockSpec((nc, bn, 1), lambda i: (0, i, 0)),
            pl.BlockSpec((bn, d), lambda i: (i, 0)),
            pl.BlockSpec((1, d), lambda i: (0, 0)),
            pl.BlockSpec((1, d), lambda i: (0, 0)),
            pl.BlockSpec((1, d), lambda i: (0, 0)),
        ],
        out_specs=pl.BlockSpec((bn, d), lambda i: (i, 0)),
        out_shape=jax.ShapeDtypeStruct((n, d), jnp.float32),
    )(part, degp, selfx, bias, gamma, beta)


def kernel(x, edge_index, edge_type, weight, alpha, bias, weight_self_loop,
           ln_gamma, ln_beta):
    n, d = x.shape
    e = edge_type.shape[0]
    r = alpha.shape[0]
    do = weight.shape[2]
    bn = 400

    nw, k, g, nsup = 32, 128, 13, 6
    epw = e // nw
    main = nsup * g * k
    row2 = edge_index[0].reshape(nw, epw)
    col2 = edge_index[1].reshape(nw, epw)
    typ2 = edge_type.reshape(nw, epw)
    rowm = row2[:, :main].reshape(nw, nsup, g, k)
    colm = col2[:, :main].reshape(nw, nsup, g, k)
    typm = typ2[:, :main].reshape(nw, nsup, g, k)
    rowt = row2[:, main:].reshape(nw, 1, epw - main)
    colt = col2[:, main:].reshape(nw, 1, epw - main)
    typt = typ2[:, main:].reshape(nw, 1, epw - main)

    h_all, selfx = _dense(x, weight, alpha, weight_self_loop, bn)
    h_flat = h_all.reshape(r * n, do)

    part, degp = _make_scatter(n, e, do)(
        rowm, colm, typm, rowt, colt, typt, h_flat)
    degp = degp.reshape(2, n)

    out = _finalize(part, degp[..., None], selfx,
                    bias.reshape(1, do), ln_gamma.reshape(1, do),
                    ln_beta.reshape(1, do), bn)
    return out


# speedup vs baseline: 36.2374x; 1.0112x over previous
import functools

import jax
import jax.numpy as jnp
from jax import lax
from jax.experimental import pallas as pl
from jax.experimental.pallas import tpu as pltpu
from jax.experimental.pallas import tpu_sc as plsc


def _dense_body(x_ref, w_ref, a_ref, h_ref):
    xb = x_ref[...]
    nb = w_ref.shape[0]
    r = h_ref.shape[0]
    for i in range(r):
        wf = a_ref[i, 0] * w_ref[0]
        for b in range(1, nb):
            wf = wf + a_ref[i, b] * w_ref[b]
        h_ref[i] = jnp.dot(xb, wf, preferred_element_type=jnp.float32)


def _dense(x, weight, alpha, bn):
    n, d = x.shape
    nb, _, do = weight.shape
    r = alpha.shape[0]
    grid = (n // bn,)
    return pl.pallas_call(
        _dense_body,
        grid=grid,
        in_specs=[
            pl.BlockSpec((bn, d), lambda i: (i, 0)),
            pl.BlockSpec((nb, d, do), lambda i: (0, 0, 0)),
            pl.BlockSpec(memory_space=pltpu.SMEM),
        ],
        out_specs=pl.BlockSpec((r, bn, do), lambda i: (0, i, 0)),
        out_shape=jax.ShapeDtypeStruct((r, n, do), jnp.float32),
    )(x, weight, alpha)


def _make_scatter(n, e, d):
    info = plsc.get_sparse_core_info()
    ncores, nsub, lanes = info.num_cores, info.num_subcores, info.num_lanes
    nw = ncores * nsub
    epw = e // nw
    k = 128
    g = 13
    nsup = 6
    tail = epw - nsup * g * k
    rpt = n // nsub
    assert tail == 16 and k % lanes == 0

    mesh = plsc.VectorSubcoreMesh(core_axis_name="c", subcore_axis_name="s")

    @functools.partial(
        pl.kernel,
        out_type=(
            jax.ShapeDtypeStruct((ncores, n, d), jnp.float32),
            jax.ShapeDtypeStruct((ncores * n,), jnp.float32),
        ),
        mesh=mesh,
        scratch_types=[
            pltpu.VMEM((1, 1, g, k), jnp.int32),
            pltpu.VMEM((1, 1, g, k), jnp.int32),
            pltpu.VMEM((1, 1, g, k), jnp.int32),
            pltpu.VMEM((1, 1, tail), jnp.int32),
            pltpu.VMEM((1, 1, tail), jnp.int32),
            pltpu.VMEM((1, 1, tail), jnp.int32),
            pltpu.VMEM((k, d), jnp.float32),
            pltpu.VMEM((k, d), jnp.float32),
            pltpu.VMEM((k,), jnp.float32),
            pltpu.VMEM((640,), jnp.float32),
            pltpu.VMEM((624,), jnp.float32),
            pltpu.VMEM_SHARED((n, d), jnp.float32),
            pltpu.VMEM_SHARED((n,), jnp.float32),
            pltpu.SemaphoreType.DMA,
            pltpu.SemaphoreType.DMA,
            pltpu.SemaphoreType.DMA,
            pltpu.SemaphoreType.DMA,
        ],
    )
    def sc_scatter(rowm_hbm, colm_hbm, typm_hbm, rowt_hbm, colt_hbm,
                   typt_hbm, h_hbm, out_hbm, deg_hbm,
                   row_all, gidx_all, col_all, row_t, gidx_t, col_t,
                   rows0, rows1, ones_buf, zvec, dbounce, accum, dega,
                   semg0, semg1, sems0, sems1):
        c = lax.axis_index("c")
        s = lax.axis_index("s")
        wid = s * ncores + c

        zero16 = jnp.zeros((lanes,), jnp.float32)
        izero16 = jnp.zeros((lanes,), jnp.int32)
        one16 = jnp.ones((lanes,), jnp.float32)
        for i in range(k // lanes):
            ones_buf[pl.ds(i * lanes, lanes)] = one16
        for i in range(k // lanes):
            row_all[0, 0, 0, pl.ds(i * lanes, lanes)] = izero16

        def zrow(i, _):
            for j in range(d // lanes):
                rows0[i, pl.ds(j * lanes, lanes)] = zero16
                rows1[i, pl.ds(j * lanes, lanes)] = zero16
            return 0
        lax.fori_loop(0, k, zrow, 0)

        def zv(i, _):
            zvec[pl.ds(i * lanes, lanes)] = zero16
            return 0
        lax.fori_loop(0, 640 // lanes, zv, 0)

        nz = rpt // k
        rem = rpt - nz * k
        for m in range(nz):
            pltpu.sync_copy(rows0, accum.at[pl.ds(s * rpt + m * k, k)])
        if rem:
            pltpu.sync_copy(rows0.at[pl.ds(0, rem)],
                            accum.at[pl.ds(s * rpt + nz * k, rem)])
        pltpu.sync_copy(zvec.at[pl.ds(0, 624)], dega.at[pl.ds(s * 624, 624)])

        @pl.when(s == 0)
        def _():
            pltpu.sync_copy(zvec.at[pl.ds(0, 16)], dega.at[pl.ds(9984, 16)])
        plsc.subcore_barrier()

        pltpu.async_copy(rows0, accum.at[row_all.at[0, 0, 0]], sems0,
                         add=True)
        pltpu.async_copy(ones_buf, dega.at[row_all.at[0, 0, 0]], sems0,
                         add=True)
        pltpu.async_copy(rows1, accum.at[row_all.at[0, 0, 0]], sems1,
                         add=True)
        pltpu.async_copy(ones_buf, dega.at[row_all.at[0, 0, 0]], sems1,
                         add=True)

        def issue(j, buf, semg, sems):
            pltpu.make_async_copy(buf, accum.at[row_all.at[0, 0, j]],
                                  sems).wait()
            pltpu.make_async_copy(ones_buf, dega.at[row_all.at[0, 0, j]],
                                  sems).wait()
            pltpu.async_copy(h_hbm.at[gidx_all.at[0, 0, j]], buf, semg)

        def consume(j, buf, semg, sems):
            pltpu.make_async_copy(h_hbm.at[gidx_all.at[0, 0, j]], buf,
                                  semg).wait()
            pltpu.async_copy(buf, accum.at[row_all.at[0, 0, j]], sems,
                             add=True)
            pltpu.async_copy(ones_buf, dega.at[row_all.at[0, 0, j]], sems,
                             add=True)

        def sup_body(sup, _):
            pltpu.sync_copy(rowm_hbm.at[pl.ds(wid, 1), pl.ds(sup, 1)],
                            row_all)
            pltpu.sync_copy(typm_hbm.at[pl.ds(wid, 1), pl.ds(sup, 1)],
                            gidx_all)
            pltpu.sync_copy(colm_hbm.at[pl.ds(wid, 1), pl.ds(sup, 1)],
                            col_all)

            def gj(j, _):
                for i in range(k // lanes):
                    sl = pl.ds(i * lanes, lanes)
                    gidx_all[0, 0, j, sl] = (gidx_all[0, 0, j, sl] * n
                                             + col_all[0, 0, j, sl])
                return 0
            lax.fori_loop(0, g, gj, 0)

            issue(0, rows0, semg0, sems0)
            issue(1, rows1, semg1, sems1)

            def pair(jj, _):
                j0 = jj * 2
                j1 = j0 + 1
                consume(j0, rows0, semg0, sems0)

                @pl.when(j0 + 2 < g)
                def _():
                    issue(j0 + 2, rows0, semg0, sems0)

                @pl.when(j1 < g)
                def _():
                    consume(j1, rows1, semg1, sems1)

                    @pl.when(j1 + 2 < g)
                    def _():
                        issue(j1 + 2, rows1, semg1, sems1)
                return 0
            lax.fori_loop(0, (g + 1) // 2, pair, 0)
            return 0
        lax.fori_loop(0, nsup, sup_body, 0)

        pltpu.make_async_copy(rows0, accum.at[row_all.at[0, 0, 0]],
                              sems0).wait()
        pltpu.make_async_copy(ones_buf, dega.at[row_all.at[0, 0, 0]],
                              sems0).wait()
        pltpu.make_async_copy(rows1, accum.at[row_all.at[0, 0, 0]],
                              sems1).wait()
        pltpu.make_async_copy(ones_buf, dega.at[row_all.at[0, 0, 0]],
                              sems1).wait()

        pltpu.sync_copy(rowt_hbm.at[pl.ds(wid, 1)], row_t)
        pltpu.sync_copy(typt_hbm.at[pl.ds(wid, 1)], gidx_t)
        pltpu.sync_copy(colt_hbm.at[pl.ds(wid, 1)], col_t)
        gidx_t[0, 0, :] = gidx_t[0, 0, :] * n + col_t[0, 0, :]
        pltpu.async_copy(h_hbm.at[gidx_t.at[0, 0]],
                         rows0.at[pl.ds(0, tail)], semg0).wait()
        pltpu.sync_copy(rows0.at[pl.ds(0, tail)],
                        accum.at[row_t.at[0, 0]], add=True)
        pltpu.sync_copy(ones_buf.at[pl.ds(0, tail)],
                        dega.at[row_t.at[0, 0]], add=True)
        plsc.subcore_barrier()

        for m in range(4):
            off = s * 624 + m * k
            buf = rows0 if m % 2 == 0 else rows1
            pltpu.sync_copy(accum.at[pl.ds(off, k)], buf)
            pltpu.sync_copy(buf, out_hbm.at[c, pl.ds(off, k)])
        off = s * 624 + 4 * k
        pltpu.sync_copy(accum.at[pl.ds(off, 112)], rows1.at[pl.ds(0, 112)])
        pltpu.sync_copy(rows1.at[pl.ds(0, 112)],
                        out_hbm.at[c, pl.ds(off, 112)])
        pltpu.sync_copy(dega.at[pl.ds(s * 624, 624)], dbounce)
        pltpu.sync_copy(dbounce, deg_hbm.at[pl.ds(c * n + s * 624, 624)])

        @pl.when(s == 0)
        def _():
            pltpu.sync_copy(accum.at[pl.ds(9984, 16)], rows0.at[pl.ds(0, 16)])
            pltpu.sync_copy(rows0.at[pl.ds(0, 16)],
                            out_hbm.at[c, pl.ds(9984, 16)])
            pltpu.sync_copy(dega.at[pl.ds(9984, 16)], dbounce.at[pl.ds(0, 16)])
            pltpu.sync_copy(dbounce.at[pl.ds(0, 16)],
                            deg_hbm.at[pl.ds(c * n + 9984, 16)])

    return sc_scatter


def _finalize_body(p_ref, dg_ref, x_ref, ws_ref, b_ref, g_ref, be_ref,
                   o_ref):
    ssum = p_ref[0] + p_ref[1]
    deg = dg_ref[0] + dg_ref[1]
    recip = jnp.where(deg > 0, 1.0 / deg, jnp.zeros_like(deg))
    h = ssum * recip
    mean = jnp.mean(h, axis=-1, keepdims=True)
    var = jnp.mean((h - mean) * (h - mean), axis=-1, keepdims=True)
    hn = (h - mean) * lax.rsqrt(var + 1e-5)
    sf = jnp.dot(x_ref[...], ws_ref[...], preferred_element_type=jnp.float32)
    o_ref[...] = hn * g_ref[...] + be_ref[...] + b_ref[...] + sf


def _finalize(part, degp, x, w_self, bias, gamma, beta, bn):
    nc, n, d = part.shape
    grid = (n // bn,)
    return pl.pallas_call(
        _finalize_body,
        grid=grid,
        in_specs=[
            pl.BlockSpec((nc, bn, d), lambda i: (0, i, 0)),
            pl.BlockSpec((nc, bn, 1), lambda i: (0, i, 0)),
            pl.BlockSpec((bn, d), lambda i: (i, 0)),
            pl.BlockSpec((d, d), lambda i: (0, 0)),
            pl.BlockSpec((1, d), lambda i: (0, 0)),
            pl.BlockSpec((1, d), lambda i: (0, 0)),
            pl.BlockSpec((1, d), lambda i: (0, 0)),
        ],
        out_specs=pl.BlockSpec((bn, d), lambda i: (i, 0)),
        out_shape=jax.ShapeDtypeStruct((n, d), jnp.float32),
    )(part, degp, x, w_self, bias, gamma, beta)


def kernel(x, edge_index, edge_type, weight, alpha, bias, weight_self_loop,
           ln_gamma, ln_beta):
    n, d = x.shape
    e = edge_type.shape[0]
    r = alpha.shape[0]
    do = weight.shape[2]
    bn = 400

    nw, k, g, nsup = 32, 128, 13, 6
    epw = e // nw
    main = nsup * g * k
    row2 = edge_index[0].reshape(nw, epw)
    col2 = edge_index[1].reshape(nw, epw)
    typ2 = edge_type.reshape(nw, epw)
    rowm = row2[:, :main].reshape(nw, nsup, g, k)
    colm = col2[:, :main].reshape(nw, nsup, g, k)
    typm = typ2[:, :main].reshape(nw, nsup, g, k)
    rowt = row2[:, main:].reshape(nw, 1, epw - main)
    colt = col2[:, main:].reshape(nw, 1, epw - main)
    typt = typ2[:, main:].reshape(nw, 1, epw - main)

    h_all = _dense(x, weight, alpha, bn)
    h_flat = h_all.reshape(r * n, do)

    part, degp = _make_scatter(n, e, do)(
        rowm, colm, typm, rowt, colt, typt, h_flat)
    degp = degp.reshape(2, n)

    out = _finalize(part, degp[..., None], x, weight_self_loop,
                    bias.reshape(1, do), ln_gamma.reshape(1, do),
                    ln_beta.reshape(1, do), bn)
    return out


# speedup vs baseline: 40.0817x; 1.1061x over previous
import functools

import jax
import jax.numpy as jnp
from jax import lax
from jax.experimental import pallas as pl
from jax.experimental.pallas import tpu as pltpu
from jax.experimental.pallas import tpu_sc as plsc


def _dense_body(x_ref, w_ref, a_ref, h_ref):
    xb = x_ref[...]
    nb = w_ref.shape[0]
    r = h_ref.shape[0]
    for i in range(r):
        wf = a_ref[i, 0] * w_ref[0]
        for b in range(1, nb):
            wf = wf + a_ref[i, b] * w_ref[b]
        h_ref[i] = jnp.dot(xb, wf, preferred_element_type=jnp.float32)


def _dense(x, weight, alpha, bn):
    n, d = x.shape
    nb, _, do = weight.shape
    r = alpha.shape[0]
    grid = (n // bn,)
    return pl.pallas_call(
        _dense_body,
        grid=grid,
        in_specs=[
            pl.BlockSpec((bn, d), lambda i: (i, 0)),
            pl.BlockSpec((nb, d, do), lambda i: (0, 0, 0)),
            pl.BlockSpec(memory_space=pltpu.SMEM),
        ],
        out_specs=pl.BlockSpec((r, bn, do), lambda i: (0, i, 0)),
        out_shape=jax.ShapeDtypeStruct((r, n, do), jnp.float32),
    )(x, weight, alpha)


def _make_deg(n, e, g, nsup):
    info = plsc.get_sparse_core_info()
    ncores, nsub, lanes = info.num_cores, info.num_subcores, info.num_lanes
    nw = ncores * nsub
    epw = e // nw
    k = 128
    tail = epw - nsup * g * k

    mesh = plsc.VectorSubcoreMesh(core_axis_name="c", subcore_axis_name="s")

    @functools.partial(
        pl.kernel,
        out_type=jax.ShapeDtypeStruct((ncores * n,), jnp.float32),
        mesh=mesh,
        scratch_types=[
            pltpu.VMEM((1, nsup, g, k), jnp.int32),
            pltpu.VMEM((1, 1, tail), jnp.int32),
            pltpu.VMEM((k,), jnp.float32),
            pltpu.VMEM((640,), jnp.float32),
            pltpu.VMEM((624,), jnp.float32),
            pltpu.VMEM_SHARED((n,), jnp.float32),
            pltpu.SemaphoreType.DMA,
        ],
    )
    def sc_deg(rowm_hbm, rowt_hbm, deg_hbm,
               row_all, row_t, ones_buf, zvec, dbounce, dega, sem):
        c = lax.axis_index("c")
        s = lax.axis_index("s")
        wid = s * ncores + c

        zero16 = jnp.zeros((lanes,), jnp.float32)
        one16 = jnp.ones((lanes,), jnp.float32)
        for i in range(k // lanes):
            ones_buf[pl.ds(i * lanes, lanes)] = one16

        def zv(i, _):
            zvec[pl.ds(i * lanes, lanes)] = zero16
            return 0
        lax.fori_loop(0, 640 // lanes, zv, 0)

        pltpu.sync_copy(zvec.at[pl.ds(0, 624)], dega.at[pl.ds(s * 624, 624)])

        @pl.when(s == 0)
        def _():
            pltpu.sync_copy(zvec.at[pl.ds(0, 16)], dega.at[pl.ds(9984, 16)])
        plsc.subcore_barrier()

        pltpu.sync_copy(rowm_hbm.at[pl.ds(wid, 1)], row_all)

        def fire_sup(sup, _):
            def fire(j, _):
                pltpu.async_copy(ones_buf, dega.at[row_all.at[0, sup, j]],
                                 sem, add=True)
                return 0
            lax.fori_loop(0, g, fire, 0)
            return 0
        lax.fori_loop(0, nsup, fire_sup, 0)

        pltpu.sync_copy(rowt_hbm.at[pl.ds(wid, 1)], row_t)
        pltpu.sync_copy(ones_buf.at[pl.ds(0, tail)],
                        dega.at[row_t.at[0, 0]], add=True)

        def drain(j, _):
            pltpu.make_async_copy(ones_buf, dega.at[row_all.at[0, 0, 0]],
                                  sem).wait()
            return 0
        lax.fori_loop(0, nsup * g, drain, 0)
        plsc.subcore_barrier()

        pltpu.sync_copy(dega.at[pl.ds(s * 624, 624)], dbounce)
        pltpu.sync_copy(dbounce, deg_hbm.at[pl.ds(c * n + s * 624, 624)])

        @pl.when(s == 0)
        def _():
            pltpu.sync_copy(dega.at[pl.ds(9984, 16)], dbounce.at[pl.ds(0, 16)])
            pltpu.sync_copy(dbounce.at[pl.ds(0, 16)],
                            deg_hbm.at[pl.ds(c * n + 9984, 16)])

    return sc_deg


def _make_scatter(n, e, d, g, nsup):
    info = plsc.get_sparse_core_info()
    ncores, nsub, lanes = info.num_cores, info.num_subcores, info.num_lanes
    nw = ncores * nsub
    epw = e // nw
    k = 128
    tail = epw - nsup * g * k
    rpt = n // nsub
    assert tail == 16 and k % lanes == 0

    mesh = plsc.VectorSubcoreMesh(core_axis_name="c", subcore_axis_name="s")

    @functools.partial(
        pl.kernel,
        out_type=jax.ShapeDtypeStruct((ncores, n, d), jnp.float32),
        mesh=mesh,
        scratch_types=[
            pltpu.VMEM((2, 1, g, k), jnp.int32),
            pltpu.VMEM((2, 1, g, k), jnp.int32),
            pltpu.VMEM((2, 1, g, k), jnp.int32),
            pltpu.VMEM((1, 1, tail), jnp.int32),
            pltpu.VMEM((1, 1, tail), jnp.int32),
            pltpu.VMEM((1, 1, tail), jnp.int32),
            pltpu.VMEM((k, d), jnp.float32),
            pltpu.VMEM((k, d), jnp.float32),
            pltpu.VMEM_SHARED((n, d), jnp.float32),
            pltpu.SemaphoreType.DMA,
            pltpu.SemaphoreType.DMA,
            pltpu.SemaphoreType.DMA,
            pltpu.SemaphoreType.DMA,
            pltpu.SemaphoreType.DMA,
            pltpu.SemaphoreType.DMA,
        ],
    )
    def sc_scatter(rowm_hbm, colm_hbm, typm_hbm, rowt_hbm, colt_hbm,
                   typt_hbm, h_hbm, out_hbm,
                   row_all, gidx_all, col_all, row_t, gidx_t, col_t,
                   rows0, rows1, accum,
                   semg0, semg1, sems0, sems1, semst, semz):
        c = lax.axis_index("c")
        s = lax.axis_index("s")
        wid = s * ncores + c

        zero16 = jnp.zeros((lanes,), jnp.float32)
        izero16 = jnp.zeros((lanes,), jnp.int32)
        for i in range(k // lanes):
            row_all[0, 0, 0, pl.ds(i * lanes, lanes)] = izero16

        def zrow(i, _):
            for j in range(d // lanes):
                rows0[i, pl.ds(j * lanes, lanes)] = zero16
                rows1[i, pl.ds(j * lanes, lanes)] = zero16
            return 0
        lax.fori_loop(0, k, zrow, 0)

        pltpu.async_copy(rows0, accum.at[row_all.at[0, 0, 0]], sems0,
                         add=True)
        pltpu.async_copy(rows1, accum.at[row_all.at[0, 0, 0]], sems1,
                         add=True)

        def stage(sup):
            st = sup % 2
            src = pl.ds(sup, 1)
            pltpu.async_copy(rowm_hbm.at[pl.ds(wid, 1), src],
                             row_all.at[pl.ds(st, 1)], semst)
            pltpu.async_copy(typm_hbm.at[pl.ds(wid, 1), src],
                             gidx_all.at[pl.ds(st, 1)], semst)
            pltpu.async_copy(colm_hbm.at[pl.ds(wid, 1), src],
                             col_all.at[pl.ds(st, 1)], semst)

        def stage_wait(sup):
            st = sup % 2
            src = pl.ds(sup, 1)
            pltpu.make_async_copy(rowm_hbm.at[pl.ds(wid, 1), src],
                                  row_all.at[pl.ds(st, 1)], semst).wait()
            pltpu.make_async_copy(typm_hbm.at[pl.ds(wid, 1), src],
                                  gidx_all.at[pl.ds(st, 1)], semst).wait()
            pltpu.make_async_copy(colm_hbm.at[pl.ds(wid, 1), src],
                                  col_all.at[pl.ds(st, 1)], semst).wait()

        stage(0)

        nz = rpt // k
        rem = rpt - nz * k
        for m in range(nz):
            pltpu.async_copy(rows0, accum.at[pl.ds(s * rpt + m * k, k)],
                             semz)
        if rem:
            pltpu.async_copy(rows0.at[pl.ds(0, rem)],
                             accum.at[pl.ds(s * rpt + nz * k, rem)], semz)
        for m in range(nz):
            pltpu.make_async_copy(rows0, accum.at[pl.ds(s * rpt + m * k, k)],
                                  semz).wait()
        if rem:
            pltpu.make_async_copy(rows0.at[pl.ds(0, rem)],
                                  accum.at[pl.ds(s * rpt + nz * k, rem)],
                                  semz).wait()
        plsc.subcore_barrier()

        def drain(buf, sems, st):
            pltpu.make_async_copy(buf, accum.at[row_all.at[st, 0, 0]],
                                  sems).wait()

        for sup in range(nsup):
            st = sup % 2
            stage_wait(sup)

            def gj(j, _, st=st):
                for i in range(k // lanes):
                    sl = pl.ds(i * lanes, lanes)
                    gidx_all[st, 0, j, sl] = (gidx_all[st, 0, j, sl] * n
                                              + col_all[st, 0, j, sl])
                return 0
            lax.fori_loop(0, g, gj, 0)

            drain(rows0, sems0, st)
            drain(rows1, sems1, st)
            if sup + 1 < nsup:
                stage(sup + 1)

            def issue(j, buf, semg, st=st):
                pltpu.async_copy(h_hbm.at[gidx_all.at[st, 0, j]], buf, semg)

            def consume(j, buf, semg, sems, st=st):
                pltpu.make_async_copy(h_hbm.at[gidx_all.at[st, 0, j]], buf,
                                      semg).wait()
                pltpu.async_copy(buf, accum.at[row_all.at[st, 0, j]], sems,
                                 add=True)

            issue(0, rows0, semg0)
            issue(1, rows1, semg1)

            def pair(jj, _):
                j0 = jj * 2
                j1 = j0 + 1
                consume(j0, rows0, semg0, sems0)

                @pl.when(j0 + 2 < g)
                def _():
                    drain(rows0, sems0, st)
                    issue(j0 + 2, rows0, semg0)

                @pl.when(j1 < g)
                def _():
                    consume(j1, rows1, semg1, sems1)

                    @pl.when(j1 + 2 < g)
                    def _():
                        drain(rows1, sems1, st)
                        issue(j1 + 2, rows1, semg1)
                return 0
            lax.fori_loop(0, (g + 1) // 2, pair, 0)

        drain(rows0, sems0, 0)
        drain(rows1, sems1, 0)

        pltpu.sync_copy(rowt_hbm.at[pl.ds(wid, 1)], row_t)
        pltpu.sync_copy(typt_hbm.at[pl.ds(wid, 1)], gidx_t)
        pltpu.sync_copy(colt_hbm.at[pl.ds(wid, 1)], col_t)
        gidx_t[0, 0, :] = gidx_t[0, 0, :] * n + col_t[0, 0, :]
        pltpu.async_copy(h_hbm.at[gidx_t.at[0, 0]],
                         rows0.at[pl.ds(0, tail)], semg0).wait()
        pltpu.sync_copy(rows0.at[pl.ds(0, tail)],
                        accum.at[row_t.at[0, 0]], add=True)
        plsc.subcore_barrier()

        for m in range(4):
            off = s * 624 + m * k
            buf = rows0 if m % 2 == 0 else rows1
            pltpu.sync_copy(accum.at[pl.ds(off, k)], buf)
            pltpu.sync_copy(buf, out_hbm.at[c, pl.ds(off, k)])
        off = s * 624 + 4 * k
        pltpu.sync_copy(accum.at[pl.ds(off, 112)], rows1.at[pl.ds(0, 112)])
        pltpu.sync_copy(rows1.at[pl.ds(0, 112)],
                        out_hbm.at[c, pl.ds(off, 112)])

        @pl.when(s == 0)
        def _():
            pltpu.sync_copy(accum.at[pl.ds(9984, 16)], rows0.at[pl.ds(0, 16)])
            pltpu.sync_copy(rows0.at[pl.ds(0, 16)],
                            out_hbm.at[c, pl.ds(9984, 16)])

    return sc_scatter


def _finalize_body(p_ref, dg_ref, x_ref, ws_ref, b_ref, g_ref, be_ref,
                   o_ref):
    ssum = p_ref[0] + p_ref[1]
    deg = dg_ref[0] + dg_ref[1]
    recip = jnp.where(deg > 0, 1.0 / deg, jnp.zeros_like(deg))
    h = ssum * recip
    mean = jnp.mean(h, axis=-1, keepdims=True)
    var = jnp.mean((h - mean) * (h - mean), axis=-1, keepdims=True)
    hn = (h - mean) * lax.rsqrt(var + 1e-5)
    sf = jnp.dot(x_ref[...], ws_ref[...], preferred_element_type=jnp.float32)
    o_ref[...] = hn * g_ref[...] + be_ref[...] + b_ref[...] + sf


def _finalize(part, degp, x, w_self, bias, gamma, beta, bn):
    nc, n, d = part.shape
    grid = (n // bn,)
    return pl.pallas_call(
        _finalize_body,
        grid=grid,
        in_specs=[
            pl.BlockSpec((nc, bn, d), lambda i: (0, i, 0)),
            pl.BlockSpec((nc, bn, 1), lambda i: (0, i, 0)),
            pl.BlockSpec((bn, d), lambda i: (i, 0)),
            pl.BlockSpec((d, d), lambda i: (0, 0)),
            pl.BlockSpec((1, d), lambda i: (0, 0)),
            pl.BlockSpec((1, d), lambda i: (0, 0)),
            pl.BlockSpec((1, d), lambda i: (0, 0)),
        ],
        out_specs=pl.BlockSpec((bn, d), lambda i: (i, 0)),
        out_shape=jax.ShapeDtypeStruct((n, d), jnp.float32),
    )(part, degp, x, w_self, bias, gamma, beta)


def kernel(x, edge_index, edge_type, weight, alpha, bias, weight_self_loop,
           ln_gamma, ln_beta):
    n, d = x.shape
    e = edge_type.shape[0]
    r = alpha.shape[0]
    do = weight.shape[2]
    bn = 400

    nw, k, g, nsup = 32, 128, 13, 6
    epw = e // nw
    main = nsup * g * k
    row2 = edge_index[0].reshape(nw, epw)
    col2 = edge_index[1].reshape(nw, epw)
    typ2 = edge_type.reshape(nw, epw)
    rowm = row2[:, :main].reshape(nw, nsup, g, k)
    colm = col2[:, :main].reshape(nw, nsup, g, k)
    typm = typ2[:, :main].reshape(nw, nsup, g, k)
    rowt = row2[:, main:].reshape(nw, 1, epw - main)
    colt = col2[:, main:].reshape(nw, 1, epw - main)
    typt = typ2[:, main:].reshape(nw, 1, epw - main)

    degp = _make_deg(n, e, g, nsup)(rowm, rowt).reshape(2, n)

    h_all = _dense(x, weight, alpha, bn)
    h_flat = h_all.reshape(r * n, do)

    part = _make_scatter(n, e, do, g, nsup)(
        rowm, colm, typm, rowt, colt, typt, h_flat)

    out = _finalize(part, degp[..., None], x, weight_self_loop,
                    bias.reshape(1, do), ln_gamma.reshape(1, do),
                    ln_beta.reshape(1, do), bn)
    return out


# speedup vs baseline: 40.1175x; 1.0009x over previous
import functools

import jax
import jax.numpy as jnp
from jax import lax
from jax.experimental import pallas as pl
from jax.experimental.pallas import tpu as pltpu
from jax.experimental.pallas import tpu_sc as plsc


def _dense_body(x_ref, w_ref, a_ref, h_ref):
    xb = x_ref[...]
    nb = w_ref.shape[0]
    r = h_ref.shape[0]
    for i in range(r):
        wf = a_ref[i, 0] * w_ref[0]
        for b in range(1, nb):
            wf = wf + a_ref[i, b] * w_ref[b]
        h_ref[i] = jnp.dot(xb, wf, preferred_element_type=jnp.float32)


def _dense(x, weight, alpha, bn):
    n, d = x.shape
    nb, _, do = weight.shape
    r = alpha.shape[0]
    grid = (n // bn,)
    return pl.pallas_call(
        _dense_body,
        grid=grid,
        in_specs=[
            pl.BlockSpec((bn, d), lambda i: (i, 0)),
            pl.BlockSpec((nb, d, do), lambda i: (0, 0, 0)),
            pl.BlockSpec(memory_space=pltpu.SMEM),
        ],
        out_specs=pl.BlockSpec((r, bn, do), lambda i: (0, i, 0)),
        out_shape=jax.ShapeDtypeStruct((r, n, do), jnp.float32),
    )(x, weight, alpha)


def _make_deg(n, e, g, nsup):
    info = plsc.get_sparse_core_info()
    ncores, nsub, lanes = info.num_cores, info.num_subcores, info.num_lanes
    nw = ncores * nsub
    epw = e // nw
    k = 128
    tail = epw - nsup * g * k

    mesh = plsc.VectorSubcoreMesh(core_axis_name="c", subcore_axis_name="s")

    @functools.partial(
        pl.kernel,
        out_type=jax.ShapeDtypeStruct((ncores * n,), jnp.float32),
        mesh=mesh,
        scratch_types=[
            pltpu.VMEM((1, nsup, g, k), jnp.int32),
            pltpu.VMEM((1, 1, tail), jnp.int32),
            pltpu.VMEM((k,), jnp.float32),
            pltpu.VMEM((640,), jnp.float32),
            pltpu.VMEM((624,), jnp.float32),
            pltpu.VMEM_SHARED((n,), jnp.float32),
            pltpu.SemaphoreType.DMA,
        ],
    )
    def sc_deg(rowm_hbm, rowt_hbm, deg_hbm,
               row_all, row_t, ones_buf, zvec, dbounce, dega, sem):
        c = lax.axis_index("c")
        s = lax.axis_index("s")
        wid = s * ncores + c

        zero16 = jnp.zeros((lanes,), jnp.float32)
        one16 = jnp.ones((lanes,), jnp.float32)
        for i in range(k // lanes):
            ones_buf[pl.ds(i * lanes, lanes)] = one16

        def zv(i, _):
            zvec[pl.ds(i * lanes, lanes)] = zero16
            return 0
        lax.fori_loop(0, 640 // lanes, zv, 0)

        pltpu.sync_copy(zvec.at[pl.ds(0, 624)], dega.at[pl.ds(s * 624, 624)])

        @pl.when(s == 0)
        def _():
            pltpu.sync_copy(zvec.at[pl.ds(0, 16)], dega.at[pl.ds(9984, 16)])
        plsc.subcore_barrier()

        pltpu.sync_copy(rowm_hbm.at[pl.ds(wid, 1)], row_all)

        def fire_sup(sup, _):
            def fire(j, _):
                pltpu.async_copy(ones_buf, dega.at[row_all.at[0, sup, j]],
                                 sem, add=True)
                return 0
            lax.fori_loop(0, g, fire, 0)
            return 0
        lax.fori_loop(0, nsup, fire_sup, 0)

        pltpu.sync_copy(rowt_hbm.at[pl.ds(wid, 1)], row_t)
        pltpu.sync_copy(ones_buf.at[pl.ds(0, tail)],
                        dega.at[row_t.at[0, 0]], add=True)

        def drain(j, _):
            pltpu.make_async_copy(ones_buf, dega.at[row_all.at[0, 0, 0]],
                                  sem).wait()
            return 0
        lax.fori_loop(0, nsup * g, drain, 0)
        plsc.subcore_barrier()

        pltpu.sync_copy(dega.at[pl.ds(s * 624, 624)], dbounce)
        pltpu.sync_copy(dbounce, deg_hbm.at[pl.ds(c * n + s * 624, 624)])

        @pl.when(s == 0)
        def _():
            pltpu.sync_copy(dega.at[pl.ds(9984, 16)], dbounce.at[pl.ds(0, 16)])
            pltpu.sync_copy(dbounce.at[pl.ds(0, 16)],
                            deg_hbm.at[pl.ds(c * n + 9984, 16)])

    return sc_deg


def _make_scatter(n, e, d, g, nsup):
    info = plsc.get_sparse_core_info()
    ncores, nsub, lanes = info.num_cores, info.num_subcores, info.num_lanes
    nw = ncores * nsub
    epw = e // nw
    k = 128
    tail = epw - nsup * g * k
    rpt = n // nsub
    assert tail == 16 and k % lanes == 0

    mesh = plsc.VectorSubcoreMesh(core_axis_name="c", subcore_axis_name="s")

    @functools.partial(
        pl.kernel,
        out_type=jax.ShapeDtypeStruct((ncores, n, d), jnp.float32),
        mesh=mesh,
        scratch_types=[
            pltpu.VMEM((2, 1, g, k), jnp.int32),
            pltpu.VMEM((2, 1, g, k), jnp.int32),
            pltpu.VMEM((2, 1, g, k), jnp.int32),
            pltpu.VMEM((1, 1, tail), jnp.int32),
            pltpu.VMEM((1, 1, tail), jnp.int32),
            pltpu.VMEM((1, 1, tail), jnp.int32),
            pltpu.VMEM((k, d), jnp.float32),
            pltpu.VMEM((k, d), jnp.float32),
            pltpu.VMEM_SHARED((n, d), jnp.float32),
            pltpu.SemaphoreType.DMA,
            pltpu.SemaphoreType.DMA,
            pltpu.SemaphoreType.DMA,
            pltpu.SemaphoreType.DMA,
            pltpu.SemaphoreType.DMA,
            pltpu.SemaphoreType.DMA,
        ],
    )
    def sc_scatter(rowm_hbm, colm_hbm, typm_hbm, rowt_hbm, colt_hbm,
                   typt_hbm, h_hbm, out_hbm,
                   row_all, gidx_all, col_all, row_t, gidx_t, col_t,
                   rows0, rows1, accum,
                   semg0, semg1, sems0, sems1, semst, semz):
        c = lax.axis_index("c")
        s = lax.axis_index("s")
        wid = s * ncores + c

        zero16 = jnp.zeros((lanes,), jnp.float32)
        izero16 = jnp.zeros((lanes,), jnp.int32)
        for i in range(k // lanes):
            row_all[0, 0, 0, pl.ds(i * lanes, lanes)] = izero16

        def zrow(i, _):
            for j in range(d // lanes):
                rows0[i, pl.ds(j * lanes, lanes)] = zero16
                rows1[i, pl.ds(j * lanes, lanes)] = zero16
            return 0
        lax.fori_loop(0, k, zrow, 0)

        pltpu.async_copy(rows0, accum.at[row_all.at[0, 0, 0]], sems0,
                         add=True)
        pltpu.async_copy(rows1, accum.at[row_all.at[0, 0, 0]], sems1,
                         add=True)

        def stage(sup):
            st = sup % 2
            src = pl.ds(sup, 1)
            pltpu.async_copy(rowm_hbm.at[pl.ds(wid, 1), src],
                             row_all.at[pl.ds(st, 1)], semst)
            pltpu.async_copy(typm_hbm.at[pl.ds(wid, 1), src],
                             gidx_all.at[pl.ds(st, 1)], semst)
            pltpu.async_copy(colm_hbm.at[pl.ds(wid, 1), src],
                             col_all.at[pl.ds(st, 1)], semst)

        def stage_wait(sup):
            st = sup % 2
            src = pl.ds(sup, 1)
            pltpu.make_async_copy(rowm_hbm.at[pl.ds(wid, 1), src],
                                  row_all.at[pl.ds(st, 1)], semst).wait()
            pltpu.make_async_copy(typm_hbm.at[pl.ds(wid, 1), src],
                                  gidx_all.at[pl.ds(st, 1)], semst).wait()
            pltpu.make_async_copy(colm_hbm.at[pl.ds(wid, 1), src],
                                  col_all.at[pl.ds(st, 1)], semst).wait()

        stage(0)

        nz = rpt // k
        rem = rpt - nz * k
        for m in range(nz):
            pltpu.async_copy(rows0, accum.at[pl.ds(s * rpt + m * k, k)],
                             semz)
        if rem:
            pltpu.async_copy(rows0.at[pl.ds(0, rem)],
                             accum.at[pl.ds(s * rpt + nz * k, rem)], semz)
        for m in range(nz):
            pltpu.make_async_copy(rows0, accum.at[pl.ds(s * rpt + m * k, k)],
                                  semz).wait()
        if rem:
            pltpu.make_async_copy(rows0.at[pl.ds(0, rem)],
                                  accum.at[pl.ds(s * rpt + nz * k, rem)],
                                  semz).wait()
        plsc.subcore_barrier()

        def drain(buf, sems, st):
            pltpu.make_async_copy(buf, accum.at[row_all.at[st, 0, 0]],
                                  sems).wait()

        for sup in range(nsup):
            st = sup % 2
            stage_wait(sup)

            def gidx(j, st=st):
                for i in range(k // lanes):
                    sl = pl.ds(i * lanes, lanes)
                    gidx_all[st, 0, j, sl] = (gidx_all[st, 0, j, sl] * n
                                              + col_all[st, 0, j, sl])

            drain(rows0, sems0, st)
            drain(rows1, sems1, st)
            if sup + 1 < nsup:
                stage(sup + 1)

            def issue(j, buf, semg, st=st):
                pltpu.async_copy(h_hbm.at[gidx_all.at[st, 0, j]], buf, semg)

            def consume(j, buf, semg, sems, st=st):
                pltpu.make_async_copy(h_hbm.at[gidx_all.at[st, 0, j]], buf,
                                      semg).wait()
                pltpu.async_copy(buf, accum.at[row_all.at[st, 0, j]], sems,
                                 add=True)

            gidx(0)
            issue(0, rows0, semg0)
            gidx(1)
            issue(1, rows1, semg1)

            def pair(jj, _):
                j0 = jj * 2
                j1 = j0 + 1

                @pl.when(j0 + 2 < g)
                def _():
                    gidx(j0 + 2)

                consume(j0, rows0, semg0, sems0)

                @pl.when(j0 + 2 < g)
                def _():
                    drain(rows0, sems0, st)
                    issue(j0 + 2, rows0, semg0)

                @pl.when(j1 < g)
                def _():
                    consume(j1, rows1, semg1, sems1)

                    @pl.when(j1 + 2 < g)
                    def _():
                        gidx(j1 + 2)
                        drain(rows1, sems1, st)
                        issue(j1 + 2, rows1, semg1)
                return 0
            lax.fori_loop(0, (g + 1) // 2, pair, 0)

        drain(rows0, sems0, 0)
        drain(rows1, sems1, 0)

        pltpu.sync_copy(rowt_hbm.at[pl.ds(wid, 1)], row_t)
        pltpu.sync_copy(typt_hbm.at[pl.ds(wid, 1)], gidx_t)
        pltpu.sync_copy(colt_hbm.at[pl.ds(wid, 1)], col_t)
        gidx_t[0, 0, :] = gidx_t[0, 0, :] * n + col_t[0, 0, :]
        pltpu.async_copy(h_hbm.at[gidx_t.at[0, 0]],
                         rows0.at[pl.ds(0, tail)], semg0).wait()
        pltpu.sync_copy(rows0.at[pl.ds(0, tail)],
                        accum.at[row_t.at[0, 0]], add=True)
        plsc.subcore_barrier()

        for m in range(4):
            off = s * 624 + m * k
            buf = rows0 if m % 2 == 0 else rows1
            pltpu.sync_copy(accum.at[pl.ds(off, k)], buf)
            pltpu.sync_copy(buf, out_hbm.at[c, pl.ds(off, k)])
        off = s * 624 + 4 * k
        pltpu.sync_copy(accum.at[pl.ds(off, 112)], rows1.at[pl.ds(0, 112)])
        pltpu.sync_copy(rows1.at[pl.ds(0, 112)],
                        out_hbm.at[c, pl.ds(off, 112)])

        @pl.when(s == 0)
        def _():
            pltpu.sync_copy(accum.at[pl.ds(9984, 16)], rows0.at[pl.ds(0, 16)])
            pltpu.sync_copy(rows0.at[pl.ds(0, 16)],
                            out_hbm.at[c, pl.ds(9984, 16)])

    return sc_scatter


def _finalize_body(p_ref, dg_ref, x_ref, ws_ref, b_ref, g_ref, be_ref,
                   o_ref):
    ssum = p_ref[0] + p_ref[1]
    deg = dg_ref[0] + dg_ref[1]
    recip = jnp.where(deg > 0, 1.0 / deg, jnp.zeros_like(deg))
    h = ssum * recip
    mean = jnp.mean(h, axis=-1, keepdims=True)
    var = jnp.mean((h - mean) * (h - mean), axis=-1, keepdims=True)
    hn = (h - mean) * lax.rsqrt(var + 1e-5)
    sf = jnp.dot(x_ref[...], ws_ref[...], preferred_element_type=jnp.float32)
    o_ref[...] = hn * g_ref[...] + be_ref[...] + b_ref[...] + sf


def _finalize(part, degp, x, w_self, bias, gamma, beta, bn):
    nc, n, d = part.shape
    grid = (n // bn,)
    return pl.pallas_call(
        _finalize_body,
        grid=grid,
        in_specs=[
            pl.BlockSpec((nc, bn, d), lambda i: (0, i, 0)),
            pl.BlockSpec((nc, bn, 1), lambda i: (0, i, 0)),
            pl.BlockSpec((bn, d), lambda i: (i, 0)),
            pl.BlockSpec((d, d), lambda i: (0, 0)),
            pl.BlockSpec((1, d), lambda i: (0, 0)),
            pl.BlockSpec((1, d), lambda i: (0, 0)),
            pl.BlockSpec((1, d), lambda i: (0, 0)),
        ],
        out_specs=pl.BlockSpec((bn, d), lambda i: (i, 0)),
        out_shape=jax.ShapeDtypeStruct((n, d), jnp.float32),
    )(part, degp, x, w_self, bias, gamma, beta)


def kernel(x, edge_index, edge_type, weight, alpha, bias, weight_self_loop,
           ln_gamma, ln_beta):
    n, d = x.shape
    e = edge_type.shape[0]
    r = alpha.shape[0]
    do = weight.shape[2]
    bn = 400

    nw, k, g, nsup = 32, 128, 13, 6
    epw = e // nw
    main = nsup * g * k
    row2 = edge_index[0].reshape(nw, epw)
    col2 = edge_index[1].reshape(nw, epw)
    typ2 = edge_type.reshape(nw, epw)
    rowm = row2[:, :main].reshape(nw, nsup, g, k)
    colm = col2[:, :main].reshape(nw, nsup, g, k)
    typm = typ2[:, :main].reshape(nw, nsup, g, k)
    rowt = row2[:, main:].reshape(nw, 1, epw - main)
    colt = col2[:, main:].reshape(nw, 1, epw - main)
    typt = typ2[:, main:].reshape(nw, 1, epw - main)

    degp = _make_deg(n, e, g, nsup)(rowm, rowt).reshape(2, n)

    h_all = _dense(x, weight, alpha, bn)
    h_flat = h_all.reshape(r * n, do)

    part = _make_scatter(n, e, do, g, nsup)(
        rowm, colm, typm, rowt, colt, typt, h_flat)

    out = _finalize(part, degp[..., None], x, weight_self_loop,
                    bias.reshape(1, do), ln_gamma.reshape(1, do),
                    ln_beta.reshape(1, do), bn)
    return out


# speedup vs baseline: 43.0635x; 1.0734x over previous
import functools

import jax
import jax.numpy as jnp
from jax import lax
from jax.experimental import pallas as pl
from jax.experimental.pallas import tpu as pltpu
from jax.experimental.pallas import tpu_sc as plsc


def _dense_body(x_ref, w_ref, a_ref, h_ref):
    xb = x_ref[...]
    nb = w_ref.shape[0]
    r = h_ref.shape[0]
    for i in range(r):
        wf = a_ref[i, 0] * w_ref[0]
        for b in range(1, nb):
            wf = wf + a_ref[i, b] * w_ref[b]
        h_ref[i] = jnp.dot(xb, wf, preferred_element_type=jnp.float32)


def _dense(x, weight, alpha, bn):
    n, d = x.shape
    nb, _, do = weight.shape
    r = alpha.shape[0]
    grid = (n // bn,)
    return pl.pallas_call(
        _dense_body,
        grid=grid,
        in_specs=[
            pl.BlockSpec((bn, d), lambda i: (i, 0)),
            pl.BlockSpec((nb, d, do), lambda i: (0, 0, 0)),
            pl.BlockSpec(memory_space=pltpu.SMEM),
        ],
        out_specs=pl.BlockSpec((r, bn, do), lambda i: (0, i, 0)),
        out_shape=jax.ShapeDtypeStruct((r, n, do), jnp.float32),
    )(x, weight, alpha)


def _make_deg(n, e, g, nsup):
    info = plsc.get_sparse_core_info()
    ncores, nsub, lanes = info.num_cores, info.num_subcores, info.num_lanes
    nw = ncores * nsub
    epw = e // nw
    k = 128
    tail = epw - nsup * g * k

    mesh = plsc.VectorSubcoreMesh(core_axis_name="c", subcore_axis_name="s")

    @functools.partial(
        pl.kernel,
        out_type=jax.ShapeDtypeStruct((ncores * n,), jnp.float32),
        mesh=mesh,
        scratch_types=[
            pltpu.VMEM((1, nsup, g, k), jnp.int32),
            pltpu.VMEM((1, 1, tail), jnp.int32),
            pltpu.VMEM((k,), jnp.float32),
            pltpu.VMEM((640,), jnp.float32),
            pltpu.VMEM((624,), jnp.float32),
            pltpu.VMEM_SHARED((n,), jnp.float32),
            pltpu.SemaphoreType.DMA,
        ],
    )
    def sc_deg(rowm_hbm, rowt_hbm, deg_hbm,
               row_all, row_t, ones_buf, zvec, dbounce, dega, sem):
        c = lax.axis_index("c")
        s = lax.axis_index("s")
        wid = s * ncores + c

        zero16 = jnp.zeros((lanes,), jnp.float32)
        one16 = jnp.ones((lanes,), jnp.float32)
        for i in range(k // lanes):
            ones_buf[pl.ds(i * lanes, lanes)] = one16

        def zv(i, _):
            zvec[pl.ds(i * lanes, lanes)] = zero16
            return 0
        lax.fori_loop(0, 640 // lanes, zv, 0)

        pltpu.sync_copy(zvec.at[pl.ds(0, 624)], dega.at[pl.ds(s * 624, 624)])

        @pl.when(s == 0)
        def _():
            pltpu.sync_copy(zvec.at[pl.ds(0, 16)], dega.at[pl.ds(9984, 16)])
        plsc.subcore_barrier()

        pltpu.sync_copy(rowm_hbm.at[pl.ds(wid, 1)], row_all)

        def fire_sup(sup, _):
            def fire(j, _):
                pltpu.async_copy(ones_buf, dega.at[row_all.at[0, sup, j]],
                                 sem, add=True)
                return 0
            lax.fori_loop(0, g, fire, 0)
            return 0
        lax.fori_loop(0, nsup, fire_sup, 0)

        pltpu.sync_copy(rowt_hbm.at[pl.ds(wid, 1)], row_t)
        pltpu.sync_copy(ones_buf.at[pl.ds(0, tail)],
                        dega.at[row_t.at[0, 0]], add=True)

        def drain(j, _):
            pltpu.make_async_copy(ones_buf, dega.at[row_all.at[0, 0, 0]],
                                  sem).wait()
            return 0
        lax.fori_loop(0, nsup * g, drain, 0)
        plsc.subcore_barrier()

        pltpu.sync_copy(dega.at[pl.ds(s * 624, 624)], dbounce)
        pltpu.sync_copy(dbounce, deg_hbm.at[pl.ds(c * n + s * 624, 624)])

        @pl.when(s == 0)
        def _():
            pltpu.sync_copy(dega.at[pl.ds(9984, 16)], dbounce.at[pl.ds(0, 16)])
            pltpu.sync_copy(dbounce.at[pl.ds(0, 16)],
                            deg_hbm.at[pl.ds(c * n + 9984, 16)])

    return sc_deg


def _make_scatter(n, e, d, g, nsup):
    info = plsc.get_sparse_core_info()
    ncores, nsub, lanes = info.num_cores, info.num_subcores, info.num_lanes
    nw = ncores * nsub
    epw = e // nw
    k = 128
    tail = epw - nsup * g * k
    rpt = n // nsub
    assert tail == 16 and k % lanes == 0

    mesh = plsc.VectorSubcoreMesh(core_axis_name="c", subcore_axis_name="s")

    @functools.partial(
        pl.kernel,
        out_type=jax.ShapeDtypeStruct((ncores, n, d), jnp.float32),
        mesh=mesh,
        scratch_types=[
            pltpu.VMEM((2, 1, g, k), jnp.int32),
            pltpu.VMEM((2, 1, g, k), jnp.int32),
            pltpu.VMEM((2, 1, g, k), jnp.int32),
            pltpu.VMEM((1, 1, tail), jnp.int32),
            pltpu.VMEM((1, 1, tail), jnp.int32),
            pltpu.VMEM((1, 1, tail), jnp.int32),
            pltpu.VMEM((k, d), jnp.float32),
            pltpu.VMEM((k, d), jnp.float32),
            pltpu.VMEM_SHARED((n, d), jnp.float32),
            pltpu.SemaphoreType.DMA,
            pltpu.SemaphoreType.DMA,
            pltpu.SemaphoreType.DMA,
            pltpu.SemaphoreType.DMA,
            pltpu.SemaphoreType.DMA,
            pltpu.SemaphoreType.DMA,
        ],
    )
    def sc_scatter(rowm_hbm, colm_hbm, typm_hbm, rowt_hbm, colt_hbm,
                   typt_hbm, h_hbm, out_hbm,
                   row_all, gidx_all, col_all, row_t, gidx_t, col_t,
                   rows0, rows1, accum,
                   semg0, semg1, sems0, sems1, semst, semz):
        c = lax.axis_index("c")
        s = lax.axis_index("s")
        wid = s * ncores + c

        zero16 = jnp.zeros((lanes,), jnp.float32)
        izero16 = jnp.zeros((lanes,), jnp.int32)
        for i in range(k // lanes):
            row_all[0, 0, 0, pl.ds(i * lanes, lanes)] = izero16

        def zrow(i, _):
            for j in range(d // lanes):
                rows0[i, pl.ds(j * lanes, lanes)] = zero16
                rows1[i, pl.ds(j * lanes, lanes)] = zero16
            return 0
        lax.fori_loop(0, k, zrow, 0)

        pltpu.async_copy(rows0, accum.at[row_all.at[0, 0, 0]], sems0,
                         add=True)
        pltpu.async_copy(rows1, accum.at[row_all.at[0, 0, 0]], sems1,
                         add=True)

        def stage(sup):
            st = sup % 2
            src = pl.ds(sup, 1)
            pltpu.async_copy(rowm_hbm.at[pl.ds(wid, 1), src],
                             row_all.at[pl.ds(st, 1)], semst)
            pltpu.async_copy(typm_hbm.at[pl.ds(wid, 1), src],
                             gidx_all.at[pl.ds(st, 1)], semst)
            pltpu.async_copy(colm_hbm.at[pl.ds(wid, 1), src],
                             col_all.at[pl.ds(st, 1)], semst)

        def stage_wait(sup):
            st = sup % 2
            src = pl.ds(sup, 1)
            pltpu.make_async_copy(rowm_hbm.at[pl.ds(wid, 1), src],
                                  row_all.at[pl.ds(st, 1)], semst).wait()
            pltpu.make_async_copy(typm_hbm.at[pl.ds(wid, 1), src],
                                  gidx_all.at[pl.ds(st, 1)], semst).wait()
            pltpu.make_async_copy(colm_hbm.at[pl.ds(wid, 1), src],
                                  col_all.at[pl.ds(st, 1)], semst).wait()

        stage(0)

        nz = rpt // k
        rem = rpt - nz * k
        for m in range(nz):
            pltpu.async_copy(rows0, accum.at[pl.ds(s * rpt + m * k, k)],
                             semz)
        if rem:
            pltpu.async_copy(rows0.at[pl.ds(0, rem)],
                             accum.at[pl.ds(s * rpt + nz * k, rem)], semz)
        for m in range(nz):
            pltpu.make_async_copy(rows0, accum.at[pl.ds(s * rpt + m * k, k)],
                                  semz).wait()
        if rem:
            pltpu.make_async_copy(rows0.at[pl.ds(0, rem)],
                                  accum.at[pl.ds(s * rpt + nz * k, rem)],
                                  semz).wait()
        plsc.subcore_barrier()

        def drain(buf, sems, st):
            pltpu.make_async_copy(buf, accum.at[row_all.at[st, 0, 0]],
                                  sems).wait()

        for sup in range(nsup):
            st = sup % 2
            stage_wait(sup)

            def gj(j, _, st=st):
                for i in range(k // lanes):
                    sl = pl.ds(i * lanes, lanes)
                    gidx_all[st, 0, j, sl] = (gidx_all[st, 0, j, sl] * n
                                              + col_all[st, 0, j, sl])
                return 0
            lax.fori_loop(0, g, gj, 0)

            drain(rows0, sems0, st)
            drain(rows1, sems1, st)
            if sup + 1 < nsup:
                stage(sup + 1)

            def issue(j, buf, semg, st=st):
                pltpu.async_copy(h_hbm.at[gidx_all.at[st, 0, j]], buf, semg)

            def consume(j, buf, semg, sems, st=st):
                pltpu.make_async_copy(h_hbm.at[gidx_all.at[st, 0, j]], buf,
                                      semg).wait()
                pltpu.async_copy(buf, accum.at[row_all.at[st, 0, j]], sems,
                                 add=True)

            issue(0, rows0, semg0)
            issue(1, rows1, semg1)

            def pair(jj, _):
                j0 = jj * 2
                j1 = j0 + 1
                consume(j0, rows0, semg0, sems0)

                @pl.when(j0 + 2 < g)
                def _():
                    drain(rows0, sems0, st)
                    issue(j0 + 2, rows0, semg0)

                @pl.when(j1 < g)
                def _():
                    consume(j1, rows1, semg1, sems1)

                    @pl.when(j1 + 2 < g)
                    def _():
                        drain(rows1, sems1, st)
                        issue(j1 + 2, rows1, semg1)
                return 0
            lax.fori_loop(0, (g + 1) // 2, pair, 0)

        drain(rows0, sems0, 0)
        drain(rows1, sems1, 0)

        pltpu.sync_copy(rowt_hbm.at[pl.ds(wid, 1)], row_t)
        pltpu.sync_copy(typt_hbm.at[pl.ds(wid, 1)], gidx_t)
        pltpu.sync_copy(colt_hbm.at[pl.ds(wid, 1)], col_t)
        gidx_t[0, 0, :] = gidx_t[0, 0, :] * n + col_t[0, 0, :]
        pltpu.async_copy(h_hbm.at[gidx_t.at[0, 0]],
                         rows0.at[pl.ds(0, tail)], semg0).wait()
        pltpu.sync_copy(rows0.at[pl.ds(0, tail)],
                        accum.at[row_t.at[0, 0]], add=True)
        plsc.subcore_barrier()

        for m in range(4):
            off = s * 624 + m * k
            buf = rows0 if m % 2 == 0 else rows1
            pltpu.sync_copy(accum.at[pl.ds(off, k)], buf)
            pltpu.sync_copy(buf, out_hbm.at[c, pl.ds(off, k)])
        off = s * 624 + 4 * k
        pltpu.sync_copy(accum.at[pl.ds(off, 112)], rows1.at[pl.ds(0, 112)])
        pltpu.sync_copy(rows1.at[pl.ds(0, 112)],
                        out_hbm.at[c, pl.ds(off, 112)])

        @pl.when(s == 0)
        def _():
            pltpu.sync_copy(accum.at[pl.ds(9984, 16)], rows0.at[pl.ds(0, 16)])
            pltpu.sync_copy(rows0.at[pl.ds(0, 16)],
                            out_hbm.at[c, pl.ds(9984, 16)])

    return sc_scatter


def _finalize_body(p_ref, dg_ref, x_ref, ws_ref, b_ref, g_ref, be_ref,
                   o_ref):
    ssum = p_ref[0] + p_ref[1]
    deg = dg_ref[0] + dg_ref[1]
    recip = jnp.where(deg > 0, 1.0 / deg, jnp.zeros_like(deg))
    h = ssum * recip
    mean = jnp.mean(h, axis=-1, keepdims=True)
    var = jnp.mean((h - mean) * (h - mean), axis=-1, keepdims=True)
    hn = (h - mean) * lax.rsqrt(var + 1e-5)
    sf = jnp.dot(x_ref[...], ws_ref[...], preferred_element_type=jnp.float32)
    o_ref[...] = hn * g_ref[...] + be_ref[...] + b_ref[...] + sf


def _finalize(part, degp, x, w_self, bias, gamma, beta, bn):
    nc, n, d = part.shape
    grid = (n // bn,)
    return pl.pallas_call(
        _finalize_body,
        grid=grid,
        in_specs=[
            pl.BlockSpec((nc, bn, d), lambda i: (0, i, 0)),
            pl.BlockSpec((nc, bn, 1), lambda i: (0, i, 0)),
            pl.BlockSpec((bn, d), lambda i: (i, 0)),
            pl.BlockSpec((d, d), lambda i: (0, 0)),
            pl.BlockSpec((1, d), lambda i: (0, 0)),
            pl.BlockSpec((1, d), lambda i: (0, 0)),
            pl.BlockSpec((1, d), lambda i: (0, 0)),
        ],
        out_specs=pl.BlockSpec((bn, d), lambda i: (i, 0)),
        out_shape=jax.ShapeDtypeStruct((n, d), jnp.float32),
    )(part, degp, x, w_self, bias, gamma, beta)


def kernel(x, edge_index, edge_type, weight, alpha, bias, weight_self_loop,
           ln_gamma, ln_beta):
    n, d = x.shape
    e = edge_type.shape[0]
    r = alpha.shape[0]
    do = weight.shape[2]
    bn = 1000

    nw, k, g, nsup = 32, 128, 13, 6
    epw = e // nw
    main = nsup * g * k
    row2 = edge_index[0].reshape(nw, epw)
    col2 = edge_index[1].reshape(nw, epw)
    typ2 = edge_type.reshape(nw, epw)
    rowm = row2[:, :main].reshape(nw, nsup, g, k)
    colm = col2[:, :main].reshape(nw, nsup, g, k)
    typm = typ2[:, :main].reshape(nw, nsup, g, k)
    rowt = row2[:, main:].reshape(nw, 1, epw - main)
    colt = col2[:, main:].reshape(nw, 1, epw - main)
    typt = typ2[:, main:].reshape(nw, 1, epw - main)

    degp = _make_deg(n, e, g, nsup)(rowm, rowt).reshape(2, n)

    h_all = _dense(x, weight, alpha, bn)
    h_flat = h_all.reshape(r * n, do)

    part = _make_scatter(n, e, do, g, nsup)(
        rowm, colm, typm, rowt, colt, typt, h_flat)

    out = _finalize(part, degp[..., None], x, weight_self_loop,
                    bias.reshape(1, do), ln_gamma.reshape(1, do),
                    ln_beta.reshape(1, do), bn)
    return out


# speedup vs baseline: 44.2511x; 1.0276x over previous
import functools

import jax
import jax.numpy as jnp
from jax import lax
from jax.experimental import pallas as pl
from jax.experimental.pallas import tpu as pltpu
from jax.experimental.pallas import tpu_sc as plsc


def _dense_body(x_ref, w_ref, a_ref, h_ref):
    xb = x_ref[...]
    nb = w_ref.shape[0]
    r = h_ref.shape[0]
    for i in range(r):
        wf = a_ref[i, 0] * w_ref[0]
        for b in range(1, nb):
            wf = wf + a_ref[i, b] * w_ref[b]
        h_ref[i] = jnp.dot(xb, wf, preferred_element_type=jnp.float32)


def _dense(x, weight, alpha, bn):
    n, d = x.shape
    nb, _, do = weight.shape
    r = alpha.shape[0]
    grid = (n // bn,)
    return pl.pallas_call(
        _dense_body,
        grid=grid,
        in_specs=[
            pl.BlockSpec((bn, d), lambda i: (i, 0)),
            pl.BlockSpec((nb, d, do), lambda i: (0, 0, 0)),
            pl.BlockSpec(memory_space=pltpu.SMEM),
        ],
        out_specs=pl.BlockSpec((r, bn, do), lambda i: (0, i, 0)),
        out_shape=jax.ShapeDtypeStruct((r, n, do), jnp.float32),
    )(x, weight, alpha)


def _make_deg(n, e, g, nsup):
    info = plsc.get_sparse_core_info()
    ncores, nsub, lanes = info.num_cores, info.num_subcores, info.num_lanes
    nw = ncores * nsub
    epw = e // nw
    k = 128
    tail = epw - nsup * g * k

    mesh = plsc.VectorSubcoreMesh(core_axis_name="c", subcore_axis_name="s")

    @functools.partial(
        pl.kernel,
        out_type=jax.ShapeDtypeStruct((ncores * n,), jnp.float32),
        mesh=mesh,
        scratch_types=[
            pltpu.VMEM((1, nsup, g, k), jnp.int32),
            pltpu.VMEM((1, 1, tail), jnp.int32),
            pltpu.VMEM((k,), jnp.float32),
            pltpu.VMEM((640,), jnp.float32),
            pltpu.VMEM((624,), jnp.float32),
            pltpu.VMEM_SHARED((n,), jnp.float32),
            pltpu.SemaphoreType.DMA,
        ],
    )
    def sc_deg(rowm_hbm, rowt_hbm, deg_hbm,
               row_all, row_t, ones_buf, zvec, dbounce, dega, sem):
        c = lax.axis_index("c")
        s = lax.axis_index("s")
        wid = s * ncores + c

        zero16 = jnp.zeros((lanes,), jnp.float32)
        one16 = jnp.ones((lanes,), jnp.float32)
        for i in range(k // lanes):
            ones_buf[pl.ds(i * lanes, lanes)] = one16

        def zv(i, _):
            zvec[pl.ds(i * lanes, lanes)] = zero16
            return 0
        lax.fori_loop(0, 640 // lanes, zv, 0)

        pltpu.sync_copy(zvec.at[pl.ds(0, 624)], dega.at[pl.ds(s * 624, 624)])

        @pl.when(s == 0)
        def _():
            pltpu.sync_copy(zvec.at[pl.ds(0, 16)], dega.at[pl.ds(9984, 16)])
        plsc.subcore_barrier()

        pltpu.sync_copy(rowm_hbm.at[pl.ds(wid, 1)], row_all)

        def fire_sup(sup, _):
            def fire(j, _):
                pltpu.async_copy(ones_buf, dega.at[row_all.at[0, sup, j]],
                                 sem, add=True)
                return 0
            lax.fori_loop(0, g, fire, 0)
            return 0
        lax.fori_loop(0, nsup, fire_sup, 0)

        pltpu.sync_copy(rowt_hbm.at[pl.ds(wid, 1)], row_t)
        pltpu.sync_copy(ones_buf.at[pl.ds(0, tail)],
                        dega.at[row_t.at[0, 0]], add=True)

        def drain(j, _):
            pltpu.make_async_copy(ones_buf, dega.at[row_all.at[0, 0, 0]],
                                  sem).wait()
            return 0
        lax.fori_loop(0, nsup * g, drain, 0)
        plsc.subcore_barrier()

        pltpu.sync_copy(dega.at[pl.ds(s * 624, 624)], dbounce)
        pltpu.sync_copy(dbounce, deg_hbm.at[pl.ds(c * n + s * 624, 624)])

        @pl.when(s == 0)
        def _():
            pltpu.sync_copy(dega.at[pl.ds(9984, 16)], dbounce.at[pl.ds(0, 16)])
            pltpu.sync_copy(dbounce.at[pl.ds(0, 16)],
                            deg_hbm.at[pl.ds(c * n + 9984, 16)])

    return sc_deg


def _make_scatter(n, e, d, g, nsup):
    info = plsc.get_sparse_core_info()
    ncores, nsub, lanes = info.num_cores, info.num_subcores, info.num_lanes
    nw = ncores * nsub
    epw = e // nw
    k = 128
    tail = epw - nsup * g * k
    rpt = n // nsub
    assert tail == 16 and k % lanes == 0

    mesh = plsc.VectorSubcoreMesh(core_axis_name="c", subcore_axis_name="s")

    @functools.partial(
        pl.kernel,
        out_type=jax.ShapeDtypeStruct((ncores, n, d), jnp.float32),
        mesh=mesh,
        scratch_types=[
            pltpu.VMEM((2, 1, g, k), jnp.int32),
            pltpu.VMEM((2, 1, g, k), jnp.int32),
            pltpu.VMEM((2, 1, g, k), jnp.int32),
            pltpu.VMEM((1, 1, tail), jnp.int32),
            pltpu.VMEM((1, 1, tail), jnp.int32),
            pltpu.VMEM((1, 1, tail), jnp.int32),
            pltpu.VMEM((k, d), jnp.float32),
            pltpu.VMEM((k, d), jnp.float32),
            pltpu.VMEM_SHARED((n, d), jnp.float32),
            pltpu.SemaphoreType.DMA,
            pltpu.SemaphoreType.DMA,
            pltpu.SemaphoreType.DMA,
            pltpu.SemaphoreType.DMA,
            pltpu.SemaphoreType.DMA,
            pltpu.SemaphoreType.DMA,
        ],
    )
    def sc_scatter(rowm_hbm, colm_hbm, typm_hbm, rowt_hbm, colt_hbm,
                   typt_hbm, h_hbm, out_hbm,
                   row_all, gidx_all, col_all, row_t, gidx_t, col_t,
                   rows0, rows1, accum,
                   semg0, semg1, sems0, sems1, semst, semz):
        c = lax.axis_index("c")
        s = lax.axis_index("s")
        wid = s * ncores + c

        zero16 = jnp.zeros((lanes,), jnp.float32)
        izero16 = jnp.zeros((lanes,), jnp.int32)
        for i in range(k // lanes):
            row_all[0, 0, 0, pl.ds(i * lanes, lanes)] = izero16

        def zrow(i, _):
            for j in range(d // lanes):
                rows0[i, pl.ds(j * lanes, lanes)] = zero16
                rows1[i, pl.ds(j * lanes, lanes)] = zero16
            return 0
        lax.fori_loop(0, k, zrow, 0)

        pltpu.async_copy(rows0, accum.at[row_all.at[0, 0, 0]], sems0,
                         add=True)
        pltpu.async_copy(rows1, accum.at[row_all.at[0, 0, 0]], sems1,
                         add=True)

        def stage(sup):
            st = sup % 2
            src = pl.ds(sup, 1)
            pltpu.async_copy(rowm_hbm.at[pl.ds(wid, 1), src],
                             row_all.at[pl.ds(st, 1)], semst)
            pltpu.async_copy(typm_hbm.at[pl.ds(wid, 1), src],
                             gidx_all.at[pl.ds(st, 1)], semst)
            pltpu.async_copy(colm_hbm.at[pl.ds(wid, 1), src],
                             col_all.at[pl.ds(st, 1)], semst)

        def stage_wait(sup):
            st = sup % 2
            src = pl.ds(sup, 1)
            pltpu.make_async_copy(rowm_hbm.at[pl.ds(wid, 1), src],
                                  row_all.at[pl.ds(st, 1)], semst).wait()
            pltpu.make_async_copy(typm_hbm.at[pl.ds(wid, 1), src],
                                  gidx_all.at[pl.ds(st, 1)], semst).wait()
            pltpu.make_async_copy(colm_hbm.at[pl.ds(wid, 1), src],
                                  col_all.at[pl.ds(st, 1)], semst).wait()

        stage(0)

        nz = rpt // k
        rem = rpt - nz * k
        for m in range(nz):
            pltpu.async_copy(rows0, accum.at[pl.ds(s * rpt + m * k, k)],
                             semz)
        if rem:
            pltpu.async_copy(rows0.at[pl.ds(0, rem)],
                             accum.at[pl.ds(s * rpt + nz * k, rem)], semz)
        for m in range(nz):
            pltpu.make_async_copy(rows0, accum.at[pl.ds(s * rpt + m * k, k)],
                                  semz).wait()
        if rem:
            pltpu.make_async_copy(rows0.at[pl.ds(0, rem)],
                                  accum.at[pl.ds(s * rpt + nz * k, rem)],
                                  semz).wait()
        plsc.subcore_barrier()

        def drain(buf, sems, st):
            pltpu.make_async_copy(buf, accum.at[row_all.at[st, 0, 0]],
                                  sems).wait()

        for sup in range(nsup):
            st = sup % 2
            stage_wait(sup)

            def gj(j, _, st=st):
                for i in range(k // lanes):
                    sl = pl.ds(i * lanes, lanes)
                    gidx_all[st, 0, j, sl] = (gidx_all[st, 0, j, sl] * n
                                              + col_all[st, 0, j, sl])
                return 0
            lax.fori_loop(0, g, gj, 0)

            drain(rows0, sems0, st)
            drain(rows1, sems1, st)
            if sup + 1 < nsup:
                stage(sup + 1)

            def issue(j, buf, semg, st=st):
                pltpu.async_copy(h_hbm.at[gidx_all.at[st, 0, j]], buf, semg)

            def consume(j, buf, semg, sems, st=st):
                pltpu.make_async_copy(h_hbm.at[gidx_all.at[st, 0, j]], buf,
                                      semg).wait()
                pltpu.async_copy(buf, accum.at[row_all.at[st, 0, j]], sems,
                                 add=True)

            issue(0, rows0, semg0)
            issue(1, rows1, semg1)

            def pair(jj, _):
                j0 = jj * 2
                j1 = j0 + 1
                consume(j0, rows0, semg0, sems0)

                @pl.when(j0 + 2 < g)
                def _():
                    drain(rows0, sems0, st)
                    issue(j0 + 2, rows0, semg0)

                @pl.when(j1 < g)
                def _():
                    consume(j1, rows1, semg1, sems1)

                    @pl.when(j1 + 2 < g)
                    def _():
                        drain(rows1, sems1, st)
                        issue(j1 + 2, rows1, semg1)
                return 0
            lax.fori_loop(0, (g + 1) // 2, pair, 0)

        drain(rows0, sems0, 0)
        drain(rows1, sems1, 0)

        pltpu.sync_copy(rowt_hbm.at[pl.ds(wid, 1)], row_t)
        pltpu.sync_copy(typt_hbm.at[pl.ds(wid, 1)], gidx_t)
        pltpu.sync_copy(colt_hbm.at[pl.ds(wid, 1)], col_t)
        gidx_t[0, 0, :] = gidx_t[0, 0, :] * n + col_t[0, 0, :]
        pltpu.async_copy(h_hbm.at[gidx_t.at[0, 0]],
                         rows0.at[pl.ds(0, tail)], semg0).wait()
        pltpu.sync_copy(rows0.at[pl.ds(0, tail)],
                        accum.at[row_t.at[0, 0]], add=True)
        plsc.subcore_barrier()

        for m in range(4):
            off = s * 624 + m * k
            buf = rows0 if m % 2 == 0 else rows1
            pltpu.sync_copy(accum.at[pl.ds(off, k)], buf)
            pltpu.sync_copy(buf, out_hbm.at[c, pl.ds(off, k)])
        off = s * 624 + 4 * k
        pltpu.sync_copy(accum.at[pl.ds(off, 112)], rows1.at[pl.ds(0, 112)])
        pltpu.sync_copy(rows1.at[pl.ds(0, 112)],
                        out_hbm.at[c, pl.ds(off, 112)])

        @pl.when(s == 0)
        def _():
            pltpu.sync_copy(accum.at[pl.ds(9984, 16)], rows0.at[pl.ds(0, 16)])
            pltpu.sync_copy(rows0.at[pl.ds(0, 16)],
                            out_hbm.at[c, pl.ds(9984, 16)])

    return sc_scatter


def _finalize_body(p_ref, dg_ref, x_ref, ws_ref, b_ref, g_ref, be_ref,
                   o_ref):
    ssum = p_ref[0] + p_ref[1]
    deg = dg_ref[0] + dg_ref[1]
    recip = jnp.where(deg > 0, 1.0 / deg, jnp.zeros_like(deg))
    h = ssum * recip
    mean = jnp.mean(h, axis=-1, keepdims=True)
    var = jnp.mean((h - mean) * (h - mean), axis=-1, keepdims=True)
    hn = (h - mean) * lax.rsqrt(var + 1e-5)
    sf = jnp.dot(x_ref[...], ws_ref[...], preferred_element_type=jnp.float32)
    o_ref[...] = hn * g_ref[...] + be_ref[...] + b_ref[...] + sf


def _finalize(part, degp, x, w_self, bias, gamma, beta, bn):
    nc, n, d = part.shape
    grid = (n // bn,)
    return pl.pallas_call(
        _finalize_body,
        grid=grid,
        in_specs=[
            pl.BlockSpec((nc, bn, d), lambda i: (0, i, 0)),
            pl.BlockSpec((nc, bn, 1), lambda i: (0, i, 0)),
            pl.BlockSpec((bn, d), lambda i: (i, 0)),
            pl.BlockSpec((d, d), lambda i: (0, 0)),
            pl.BlockSpec((1, d), lambda i: (0, 0)),
            pl.BlockSpec((1, d), lambda i: (0, 0)),
            pl.BlockSpec((1, d), lambda i: (0, 0)),
        ],
        out_specs=pl.BlockSpec((bn, d), lambda i: (i, 0)),
        out_shape=jax.ShapeDtypeStruct((n, d), jnp.float32),
    )(part, degp, x, w_self, bias, gamma, beta)


def kernel(x, edge_index, edge_type, weight, alpha, bias, weight_self_loop,
           ln_gamma, ln_beta):
    n, d = x.shape
    e = edge_type.shape[0]
    r = alpha.shape[0]
    do = weight.shape[2]
    bn = 2000

    nw, k, g, nsup = 32, 128, 13, 6
    epw = e // nw
    main = nsup * g * k
    row2 = edge_index[0].reshape(nw, epw)
    col2 = edge_index[1].reshape(nw, epw)
    typ2 = edge_type.reshape(nw, epw)
    rowm = row2[:, :main].reshape(nw, nsup, g, k)
    colm = col2[:, :main].reshape(nw, nsup, g, k)
    typm = typ2[:, :main].reshape(nw, nsup, g, k)
    rowt = row2[:, main:].reshape(nw, 1, epw - main)
    colt = col2[:, main:].reshape(nw, 1, epw - main)
    typt = typ2[:, main:].reshape(nw, 1, epw - main)

    degp = _make_deg(n, e, g, nsup)(rowm, rowt).reshape(2, n)

    h_all = _dense(x, weight, alpha, bn)
    h_flat = h_all.reshape(r * n, do)

    part = _make_scatter(n, e, do, g, nsup)(
        rowm, colm, typm, rowt, colt, typt, h_flat)

    out = _finalize(part, degp[..., None], x, weight_self_loop,
                    bias.reshape(1, do), ln_gamma.reshape(1, do),
                    ln_beta.reshape(1, do), bn)
    return out
